# baseline (speedup 1.0000x reference)
import functools

import jax
import jax.numpy as jnp
from jax import lax
from jax.experimental import pallas as pl
from jax.experimental.pallas import tpu as pltpu
from jax.experimental.pallas import tpu_sc as plsc

F32 = jnp.float32
U32 = jnp.uint32


def _pack_bf16_pairs(y):
    yi = lax.bitcast_convert_type(y, U32)
    lsb = (yi >> 16) & jnp.uint32(1)
    r = (yi + jnp.uint32(0x7FFF) + lsb) >> 16
    half = y.shape[-1] // 2
    pk = r[:, :half] | (r[:, half:] << 16)
    return lax.bitcast_convert_type(pk, jnp.int32)


def _encode(x, w_t, b, block_rows=2000):
    M, K = x.shape
    Hout = w_t.shape[1]
    nb = M // block_rows

    def body(x_ref, w_ref, b_ref, o_ref, op_ref):
        y = jnp.dot(x_ref[...], w_ref[...], preferred_element_type=F32) + b_ref[...]
        o_ref[...] = y
        op_ref[...] = _pack_bf16_pairs(y)

    full = lambda i: (0, 0)
    return pl.pallas_call(
        body,
        grid=(nb,),
        in_specs=[
            pl.BlockSpec((block_rows, K), lambda i: (i, 0)),
            pl.BlockSpec((K, Hout), full),
            pl.BlockSpec((1, Hout), full),
        ],
        out_specs=[
            pl.BlockSpec((block_rows, Hout), lambda i: (i, 0)),
            pl.BlockSpec((block_rows, Hout // 2), lambda i: (i, 0)),
        ],
        out_shape=[
            jax.ShapeDtypeStruct((M, Hout), F32),
            jax.ShapeDtypeStruct((M, Hout // 2), jnp.int32),
        ],
    )(x, w_t, b)


def _edge_embed(attr, w0_t, b0, w1_t, b1, block_rows=2000):
    E, D = attr.shape
    H = w0_t.shape[1]
    nb = E // block_rows

    def body(a_ref, w0_ref, b0_ref, w1_ref, b1_ref, e0_ref, e1_ref):
        a = a_ref[...]
        e0_ref[...] = _pack_bf16_pairs(
            jnp.dot(a, w0_ref[...], preferred_element_type=F32) + b0_ref[...]
        )
        e1_ref[...] = _pack_bf16_pairs(
            jnp.dot(a, w1_ref[...], preferred_element_type=F32) + b1_ref[...]
        )

    full = lambda i: (0, 0)
    return pl.pallas_call(
        body,
        grid=(nb,),
        in_specs=[
            pl.BlockSpec((block_rows, D), lambda i: (i, 0)),
            pl.BlockSpec((D, H), full),
            pl.BlockSpec((1, H), full),
            pl.BlockSpec((D, H), full),
            pl.BlockSpec((1, H), full),
        ],
        out_specs=[
            pl.BlockSpec((block_rows, H // 2), lambda i: (i, 0)),
            pl.BlockSpec((block_rows, H // 2), lambda i: (i, 0)),
        ],
        out_shape=[
            jax.ShapeDtypeStruct((E, H // 2), jnp.int32),
            jax.ShapeDtypeStruct((E, H // 2), jnp.int32),
        ],
    )(attr, w0_t, b0, w1_t, b1)


def _post(x, agg, w1_t, b1, w2_t, b2, g, b, emit_packed, block_rows=2000):
    M, H = x.shape
    nb = M // block_rows
    full = lambda i: (0, 0)
    blk = lambda i: (i, 0)

    def body(x_ref, a_ref, w1_ref, b1_ref, w2_ref, b2_ref, g_ref, bb_ref,
             o_ref, *op_ref):
        x_ = x_ref[...]
        h = x_ + a_ref[...]
        y = jnp.maximum(
            jnp.dot(h, w1_ref[...], preferred_element_type=F32) + b1_ref[...], 0.0
        )
        y = jnp.dot(y, w2_ref[...], preferred_element_type=F32) + b2_ref[...]
        mu = jnp.mean(y, axis=-1, keepdims=True)
        var = jnp.mean((y - mu) ** 2, axis=-1, keepdims=True)
        z = (y - mu) * lax.rsqrt(var + 1e-5) * g_ref[...] + bb_ref[...]
        o = jnp.maximum(z, 0.0) + x_
        o_ref[...] = o
        if op_ref:
            op_ref[0][...] = _pack_bf16_pairs(o)

    out_specs = [pl.BlockSpec((block_rows, H), blk)]
    out_shape = [jax.ShapeDtypeStruct((M, H), F32)]
    if emit_packed:
        out_specs.append(pl.BlockSpec((block_rows, H // 2), blk))
        out_shape.append(jax.ShapeDtypeStruct((M, H // 2), jnp.int32))

    res = pl.pallas_call(
        body,
        grid=(nb,),
        in_specs=[pl.BlockSpec((block_rows, H), blk)] * 2
        + [
            pl.BlockSpec((H, H), full),
            pl.BlockSpec((1, H), full),
            pl.BlockSpec((H, H), full),
            pl.BlockSpec((1, H), full),
            pl.BlockSpec((1, H), full),
            pl.BlockSpec((1, H), full),
        ],
        out_specs=out_specs,
        out_shape=out_shape,
    )(x, agg, w1_t, b1, w2_t, b2, g, b)
    return res if emit_packed else (res[0], None)


def _gru_head(seq, wih_t, whh_t, bih, bhh, w_head, b_head, block_rows=1024):
    T, Np, H = seq.shape
    nb = Np // block_rows

    def body(s_ref, wih_ref, whh_ref, bih_ref, bhh_ref, wh_ref, bh_ref, o_ref):
        h = jnp.zeros((block_rows, H), F32)
        for t in range(T):
            xt = s_ref[t]
            gx = jnp.dot(xt, wih_ref[...], preferred_element_type=F32) + bih_ref[...]
            gh = jnp.dot(h, whh_ref[...], preferred_element_type=F32) + bhh_ref[...]
            r = jax.nn.sigmoid(gx[:, :H] + gh[:, :H])
            z = jax.nn.sigmoid(gx[:, H : 2 * H] + gh[:, H : 2 * H])
            n = jnp.tanh(gx[:, 2 * H :] + r * gh[:, 2 * H :])
            h = (1.0 - z) * n + z * h
        p = jnp.sum(h * wh_ref[...], axis=1, keepdims=True) + bh_ref[0, 0]
        o_ref[...] = jnp.broadcast_to(p, (block_rows, H))

    full = lambda i: (0, 0)
    return pl.pallas_call(
        body,
        grid=(nb,),
        in_specs=[
            pl.BlockSpec((T, block_rows, H), lambda i: (0, i, 0)),
            pl.BlockSpec((H, 3 * H), full),
            pl.BlockSpec((H, 3 * H), full),
            pl.BlockSpec((1, 3 * H), full),
            pl.BlockSpec((1, 3 * H), full),
            pl.BlockSpec((1, H), full),
            pl.BlockSpec((1, 1), full),
        ],
        out_specs=pl.BlockSpec((block_rows, H), lambda i: (i, 0)),
        out_shape=jax.ShapeDtypeStruct((Np, H), F32),
    )(seq, wih_t, whh_t, bih, bhh, w_head, b_head)


def _message(x_pk, src, dst, e_pk, zeros_blk, T, N, H):
    E = src.shape[0]
    HW = H // 2
    NSUB = 16
    NCORE = 2
    TP = T // NCORE
    CH = 80
    GCH = E // CH
    NCH = (GCH + NSUB - 1) // NSUB
    NCH = NCH + (NCH & 1)
    NP = ((N + 127) // 128) * 128
    NPS = NP // NSUB
    DUMP = NP - 8

    mesh = plsc.VectorSubcoreMesh(
        core_axis_name="c", subcore_axis_name="s", num_cores=NCORE, num_subcores=NSUB
    )

    @functools.partial(
        pl.kernel,
        out_type=jax.ShapeDtypeStruct((T * NP, H), F32),
        mesh=mesh,
        compiler_params=pltpu.CompilerParams(
            needs_layout_passes=False, use_tc_tiling_on_sc=False
        ),
        scratch_types=[
            [pltpu.VMEM((CH,), jnp.int32)] * 2,
            [pltpu.VMEM((CH,), jnp.int32)] * 2,
            [pltpu.VMEM((CH,), jnp.int32)] * 2,
            [pltpu.VMEM((CH, HW), jnp.int32)] * 2,
            [pltpu.VMEM((CH, HW), jnp.int32)] * 2,
            [pltpu.VMEM((CH, H), F32)] * 2,
            pltpu.VMEM_SHARED((NP, H), F32),
            [pltpu.SemaphoreType.DMA] * 2,
            [pltpu.SemaphoreType.DMA] * 2,
            [pltpu.SemaphoreType.DMA] * 2,
        ],
    )
    def msg(x_hbm, src_hbm, dst_hbm, e_hbm, z_hbm, out_hbm, src_v, dst_v, dsc_v,
            e_v, xr_v, xf_v, acc, fsem, gsem, ssem):
        c = lax.axis_index("c")
        s = lax.axis_index("s")

        def chunk_off(k):
            g = k * NSUB + s
            g = jnp.minimum(g, GCH - 1)
            return g * CH

        def fetch(k, b):
            off = chunk_off(k)
            pltpu.async_copy(src_hbm.at[pl.ds(off, CH)], src_v[b], fsem[b])
            pltpu.async_copy(dst_hbm.at[pl.ds(off, CH)], dst_v[b], fsem[b])
            pltpu.async_copy(e_hbm.at[pl.ds(off, CH)], e_v[b], fsem[b])

        def wait_fetch(b):
            pltpu.make_async_copy(src_hbm.at[pl.ds(0, CH)], src_v[b], fsem[b]).wait()
            pltpu.make_async_copy(dst_hbm.at[pl.ds(0, CH)], dst_v[b], fsem[b]).wait()
            pltpu.make_async_copy(e_hbm.at[pl.ds(0, CH)], e_v[b], fsem[b]).wait()

        def prep_idx(k, b, t):
            fake = (k * NSUB + s) >= GCH
            for g in range(CH // 16):
                sl = pl.ds(g * 16, 16)
                src_v[b][sl] = src_v[b][sl] + t * N
                dst_v[b][sl] = jnp.where(fake, DUMP, dst_v[b][sl])

        def gather(b):
            pltpu.async_copy(x_hbm.at[src_v[b]], xr_v[b], gsem[b])

        def wait_gather(b):
            pltpu.make_async_copy(x_hbm.at[src_v[b]], xr_v[b], gsem[b]).wait()

        def wait_scatter(b):
            pltpu.make_async_copy(xf_v[b], acc.at[dsc_v[b]], ssem[b]).wait()

        for j in range(TP):
            t = c * TP + j
            pltpu.sync_copy(z_hbm, acc.at[pl.ds(s * NPS, NPS)])
            plsc.subcore_barrier()

            fetch(0, 0)
            fetch(1, 1)
            wait_fetch(0)
            prep_idx(0, 0, t)
            gather(0)

            def step(ci2, _):
                for b in (0, 1):
                    ci = ci2 * 2 + b
                    nb = 1 - b

                    @pl.when(ci + 1 < NCH)
                    def _():
                        wait_fetch(nb)
                        prep_idx(ci + 1, nb, t)

                    @pl.when(ci >= 1)
                    def _():
                        wait_scatter(nb)

                    @pl.when(ci + 1 < NCH)
                    def _():
                        gather(nb)

                    wait_gather(b)

                    def row(i, _):
                        for g in range(HW // 16):
                            sl = pl.ds(g * 16, 16)
                            xw = xr_v[b][i, sl]
                            ew = e_v[b][i, sl]
                            xlo = plsc.bitcast(xw << 16, F32)
                            xhi = plsc.bitcast(xw & jnp.int32(-65536), F32)
                            elo = plsc.bitcast(ew << 16, F32)
                            ehi = plsc.bitcast(ew & jnp.int32(-65536), F32)
                            xf_v[b][i, pl.ds(g * 16, 16)] = jnp.maximum(
                                xlo + elo, 0.0
                            )
                            xf_v[b][i, pl.ds(HW + g * 16, 16)] = jnp.maximum(
                                xhi + ehi, 0.0
                            )
                        return 0

                    lax.fori_loop(0, CH, row, 0)
                    for g in range(CH // 16):
                        sl = pl.ds(g * 16, 16)
                        dsc_v[b][sl] = dst_v[b][sl]
                    pltpu.async_copy(xf_v[b], acc.at[dsc_v[b]], ssem[b], add=True)

                    @pl.when(ci + 2 < NCH)
                    def _():
                        fetch(ci + 2, b)

                return 0

            lax.fori_loop(0, NCH // 2, step, 0)
            wait_scatter(1)
            plsc.subcore_barrier()
            pltpu.sync_copy(
                acc.at[pl.ds(s * NPS, NPS)],
                out_hbm.at[pl.ds(t * NP + s * NPS, NPS)],
            )

    NPpad = ((N + 127) // 128) * 128
    out = msg(x_pk, src, dst, e_pk, zeros_blk)
    return out.reshape(T, NPpad, H)[:, :N].reshape(T * N, H)


def kernel(x_seq, edge_index, edge_attr, W_enc, b_enc, lin0_W, lin0_b, mlp0_W1,
           mlp0_b1, mlp0_W2, mlp0_b2, ln0_g, ln0_b, lin1_W, lin1_b, mlp1_W1,
           mlp1_b1, mlp1_W2, mlp1_b2, ln1_g, ln1_b, W_ih, W_hh, b_ih, b_hh,
           W_head, b_head):
    B, T, N, F = x_seq.shape
    H = W_enc.shape[0]
    src = edge_index[0]
    dst = edge_index[1]

    r2 = lambda v: v.reshape(1, -1)

    e0p, e1p = _edge_embed(edge_attr, lin0_W.T, r2(lin0_b), lin1_W.T, r2(lin1_b))
    X, Xp = _encode(x_seq.reshape(T * N, F), W_enc.T, r2(b_enc))

    zeros_blk = jnp.zeros((((N + 127) // 128) * 128 // 16, H), F32)

    agg0 = _message(Xp, src, dst, e0p, zeros_blk, T, N, H)
    X1, X1p = _post(X, agg0, mlp0_W1.T, r2(mlp0_b1), mlp0_W2.T, r2(mlp0_b2),
                    r2(ln0_g), r2(ln0_b), emit_packed=True)
    agg1 = _message(X1p, src, dst, e1p, zeros_blk, T, N, H)
    X2, _ = _post(X1, agg1, mlp1_W1.T, r2(mlp1_b1), mlp1_W2.T, r2(mlp1_b2),
                  r2(ln1_g), r2(ln1_b), emit_packed=False)

    Np = ((N + 1023) // 1024) * 1024
    seq = jnp.pad(X2.reshape(T, N, H), ((0, 0), (0, Np - N), (0, 0)))
    hout = _gru_head(seq, W_ih.T, W_hh.T, r2(b_ih), r2(b_hh), W_head,
                     b_head.reshape(1, 1))
    return hout[:N, 0].reshape(1, N)


# speedup vs baseline: 1.7754x; 1.7754x over previous
import functools

import jax
import jax.numpy as jnp
from jax import lax
from jax.experimental import pallas as pl
from jax.experimental.pallas import tpu as pltpu
from jax.experimental.pallas import tpu_sc as plsc

F32 = jnp.float32
U32 = jnp.uint32


def _pack_bf16_pairs(y):
    yi = lax.bitcast_convert_type(y, U32)
    lsb = (yi >> 16) & jnp.uint32(1)
    r = (yi + jnp.uint32(0x7FFF) + lsb) >> 16
    half = y.shape[-1] // 2
    pk = r[:, :half] | (r[:, half:] << 16)
    return lax.bitcast_convert_type(pk, jnp.int32)


def _encode(x, w_t, b, block_rows=2000):
    M, K = x.shape
    Hout = w_t.shape[1]
    nb = M // block_rows

    def body(x_ref, w_ref, b_ref, o_ref, op_ref):
        y = jnp.dot(x_ref[...], w_ref[...], preferred_element_type=F32) + b_ref[...]
        o_ref[...] = y
        op_ref[...] = _pack_bf16_pairs(y)

    full = lambda i: (0, 0)
    return pl.pallas_call(
        body,
        grid=(nb,),
        in_specs=[
            pl.BlockSpec((block_rows, K), lambda i: (i, 0)),
            pl.BlockSpec((K, Hout), full),
            pl.BlockSpec((1, Hout), full),
        ],
        out_specs=[
            pl.BlockSpec((block_rows, Hout), lambda i: (i, 0)),
            pl.BlockSpec((block_rows, Hout // 2), lambda i: (i, 0)),
        ],
        out_shape=[
            jax.ShapeDtypeStruct((M, Hout), F32),
            jax.ShapeDtypeStruct((M, Hout // 2), jnp.int32),
        ],
    )(x, w_t, b)


def _edge_embed(attr, w0_t, b0, w1_t, b1, block_rows=2000):
    E, D = attr.shape
    H = w0_t.shape[1]
    nb = E // block_rows

    def body(a_ref, w0_ref, b0_ref, w1_ref, b1_ref, e0_ref, e1_ref):
        a = a_ref[...]
        e0_ref[...] = _pack_bf16_pairs(
            jnp.dot(a, w0_ref[...], preferred_element_type=F32) + b0_ref[...]
        )
        e1_ref[...] = _pack_bf16_pairs(
            jnp.dot(a, w1_ref[...], preferred_element_type=F32) + b1_ref[...]
        )

    full = lambda i: (0, 0)
    return pl.pallas_call(
        body,
        grid=(nb,),
        in_specs=[
            pl.BlockSpec((block_rows, D), lambda i: (i, 0)),
            pl.BlockSpec((D, H), full),
            pl.BlockSpec((1, H), full),
            pl.BlockSpec((D, H), full),
            pl.BlockSpec((1, H), full),
        ],
        out_specs=[
            pl.BlockSpec((block_rows, H // 2), lambda i: (i, 0)),
            pl.BlockSpec((block_rows, H // 2), lambda i: (i, 0)),
        ],
        out_shape=[
            jax.ShapeDtypeStruct((E, H // 2), jnp.int32),
            jax.ShapeDtypeStruct((E, H // 2), jnp.int32),
        ],
    )(attr, w0_t, b0, w1_t, b1)


def _post(x, agg, w1_t, b1, w2_t, b2, g, b, emit_packed, block_rows=2000):
    M, H = x.shape
    nb = M // block_rows
    full = lambda i: (0, 0)
    blk = lambda i: (i, 0)

    def body(x_ref, a_ref, w1_ref, b1_ref, w2_ref, b2_ref, g_ref, bb_ref,
             o_ref, *op_ref):
        x_ = x_ref[...]
        h = x_ + a_ref[...]
        y = jnp.maximum(
            jnp.dot(h, w1_ref[...], preferred_element_type=F32) + b1_ref[...], 0.0
        )
        y = jnp.dot(y, w2_ref[...], preferred_element_type=F32) + b2_ref[...]
        mu = jnp.mean(y, axis=-1, keepdims=True)
        var = jnp.mean((y - mu) ** 2, axis=-1, keepdims=True)
        z = (y - mu) * lax.rsqrt(var + 1e-5) * g_ref[...] + bb_ref[...]
        o = jnp.maximum(z, 0.0) + x_
        o_ref[...] = o
        if op_ref:
            op_ref[0][...] = _pack_bf16_pairs(o)

    out_specs = [pl.BlockSpec((block_rows, H), blk)]
    out_shape = [jax.ShapeDtypeStruct((M, H), F32)]
    if emit_packed:
        out_specs.append(pl.BlockSpec((block_rows, H // 2), blk))
        out_shape.append(jax.ShapeDtypeStruct((M, H // 2), jnp.int32))

    res = pl.pallas_call(
        body,
        grid=(nb,),
        in_specs=[pl.BlockSpec((block_rows, H), blk)] * 2
        + [
            pl.BlockSpec((H, H), full),
            pl.BlockSpec((1, H), full),
            pl.BlockSpec((H, H), full),
            pl.BlockSpec((1, H), full),
            pl.BlockSpec((1, H), full),
            pl.BlockSpec((1, H), full),
        ],
        out_specs=out_specs,
        out_shape=out_shape,
    )(x, agg, w1_t, b1, w2_t, b2, g, b)
    return res if emit_packed else (res[0], None)


def _gru_head(seq, wih_t, whh_t, bih, bhh, w_head, b_head, block_rows=1024):
    T, Np, H = seq.shape
    nb = Np // block_rows

    def body(s_ref, wih_ref, whh_ref, bih_ref, bhh_ref, wh_ref, bh_ref, o_ref):
        h = jnp.zeros((block_rows, H), F32)
        for t in range(T):
            xt = s_ref[t]
            gx = jnp.dot(xt, wih_ref[...], preferred_element_type=F32) + bih_ref[...]
            gh = jnp.dot(h, whh_ref[...], preferred_element_type=F32) + bhh_ref[...]
            r = jax.nn.sigmoid(gx[:, :H] + gh[:, :H])
            z = jax.nn.sigmoid(gx[:, H : 2 * H] + gh[:, H : 2 * H])
            n = jnp.tanh(gx[:, 2 * H :] + r * gh[:, 2 * H :])
            h = (1.0 - z) * n + z * h
        p = jnp.sum(h * wh_ref[...], axis=1, keepdims=True) + bh_ref[0, 0]
        o_ref[...] = jnp.broadcast_to(p, (block_rows, H))

    full = lambda i: (0, 0)
    return pl.pallas_call(
        body,
        grid=(nb,),
        in_specs=[
            pl.BlockSpec((T, block_rows, H), lambda i: (0, i, 0)),
            pl.BlockSpec((H, 3 * H), full),
            pl.BlockSpec((H, 3 * H), full),
            pl.BlockSpec((1, 3 * H), full),
            pl.BlockSpec((1, 3 * H), full),
            pl.BlockSpec((1, H), full),
            pl.BlockSpec((1, 1), full),
        ],
        out_specs=pl.BlockSpec((block_rows, H), lambda i: (i, 0)),
        out_shape=jax.ShapeDtypeStruct((Np, H), F32),
    )(seq, wih_t, whh_t, bih, bhh, w_head, b_head)


def _message(x_pk, src, dst, e_pk, zeros_blk, T, N, H):
    E = src.shape[0]
    HW = H // 2
    NSUB = 16
    NCORE = 2
    TP = T // NCORE
    CH = 80
    GCH = E // CH
    NCH = (GCH + NSUB - 1) // NSUB
    NCH = NCH + (NCH & 1)
    NP = ((N + 127) // 128) * 128
    NPS = NP // NSUB
    DUMP = NP - 8

    mesh = plsc.VectorSubcoreMesh(
        core_axis_name="c", subcore_axis_name="s", num_cores=NCORE, num_subcores=NSUB
    )

    @functools.partial(
        pl.kernel,
        out_type=jax.ShapeDtypeStruct((T * NP, H), F32),
        mesh=mesh,
        compiler_params=pltpu.CompilerParams(
            needs_layout_passes=False, use_tc_tiling_on_sc=False
        ),
        scratch_types=[
            [pltpu.VMEM((CH,), jnp.int32)] * 2,
            [pltpu.VMEM((CH,), jnp.int32)] * 2,
            [pltpu.VMEM((CH,), jnp.int32)] * 2,
            [pltpu.VMEM((CH, HW), jnp.int32)] * 2,
            [pltpu.VMEM((CH, HW), jnp.int32)] * 2,
            [pltpu.VMEM((CH, H), F32)] * 2,
            pltpu.VMEM_SHARED((NP, H), F32),
            [pltpu.SemaphoreType.DMA] * 2,
            [pltpu.SemaphoreType.DMA] * 2,
            [pltpu.SemaphoreType.DMA] * 2,
        ],
    )
    def msg(x_hbm, src_hbm, dst_hbm, e_hbm, z_hbm, out_hbm, src_v, dst_v, dsc_v,
            e_v, xr_v, xf_v, acc, fsem, gsem, ssem):
        c = lax.axis_index("c")
        s = lax.axis_index("s")

        def chunk_off(k):
            g = k * NSUB + s
            g = jnp.minimum(g, GCH - 1)
            return g * CH

        def fetch(k, b):
            off = chunk_off(k)
            pltpu.async_copy(src_hbm.at[pl.ds(off, CH)], src_v[b], fsem[b])
            pltpu.async_copy(dst_hbm.at[pl.ds(off, CH)], dst_v[b], fsem[b])
            pltpu.async_copy(e_hbm.at[pl.ds(off, CH)], e_v[b], fsem[b])

        def wait_fetch(b):
            pltpu.make_async_copy(src_hbm.at[pl.ds(0, CH)], src_v[b], fsem[b]).wait()
            pltpu.make_async_copy(dst_hbm.at[pl.ds(0, CH)], dst_v[b], fsem[b]).wait()
            pltpu.make_async_copy(e_hbm.at[pl.ds(0, CH)], e_v[b], fsem[b]).wait()

        def prep_idx(k, b, t):
            fake = (k * NSUB + s) >= GCH
            for g in range(CH // 16):
                sl = pl.ds(g * 16, 16)
                src_v[b][sl] = src_v[b][sl] + t * N
                dst_v[b][sl] = jnp.where(fake, DUMP, dst_v[b][sl])

        def gather(b):
            pltpu.async_copy(x_hbm.at[src_v[b]], xr_v[b], gsem[b])

        def wait_gather(b):
            pltpu.make_async_copy(x_hbm.at[src_v[b]], xr_v[b], gsem[b]).wait()

        def wait_scatter(b):
            pltpu.make_async_copy(xf_v[b], acc.at[dsc_v[b]], ssem[b]).wait()

        for j in range(TP):
            t = c * TP + j
            pltpu.sync_copy(z_hbm, acc.at[pl.ds(s * NPS, NPS)])
            plsc.subcore_barrier()

            fetch(0, 0)
            fetch(1, 1)
            wait_fetch(0)
            prep_idx(0, 0, t)
            gather(0)

            def step(ci2, _):
                for b in (0, 1):
                    ci = ci2 * 2 + b
                    nb = 1 - b

                    @pl.when(ci + 1 < NCH)
                    def _():
                        wait_fetch(nb)
                        prep_idx(ci + 1, nb, t)

                    @pl.when(ci >= 1)
                    def _():
                        wait_scatter(nb)

                    @pl.when(ci + 1 < NCH)
                    def _():
                        gather(nb)

                    wait_gather(b)

                    @plsc.parallel_loop(0, CH, unroll=4)
                    def row(i):
                        for g in range(HW // 16):
                            sl = pl.ds(g * 16, 16)
                            xw = xr_v[b][i, sl]
                            ew = e_v[b][i, sl]
                            xlo = plsc.bitcast(xw << 16, F32)
                            xhi = plsc.bitcast(xw & jnp.int32(-65536), F32)
                            elo = plsc.bitcast(ew << 16, F32)
                            ehi = plsc.bitcast(ew & jnp.int32(-65536), F32)
                            xf_v[b][i, pl.ds(g * 16, 16)] = jnp.maximum(
                                xlo + elo, 0.0
                            )
                            xf_v[b][i, pl.ds(HW + g * 16, 16)] = jnp.maximum(
                                xhi + ehi, 0.0
                            )
                    for g in range(CH // 16):
                        sl = pl.ds(g * 16, 16)
                        dsc_v[b][sl] = dst_v[b][sl]
                    pltpu.async_copy(xf_v[b], acc.at[dsc_v[b]], ssem[b], add=True)

                    @pl.when(ci + 2 < NCH)
                    def _():
                        fetch(ci + 2, b)

                return 0

            lax.fori_loop(0, NCH // 2, step, 0)
            wait_scatter(1)
            plsc.subcore_barrier()
            pltpu.sync_copy(
                acc.at[pl.ds(s * NPS, NPS)],
                out_hbm.at[pl.ds(t * NP + s * NPS, NPS)],
            )

    NPpad = ((N + 127) // 128) * 128
    out = msg(x_pk, src, dst, e_pk, zeros_blk)
    return out.reshape(T, NPpad, H)[:, :N].reshape(T * N, H)


def kernel(x_seq, edge_index, edge_attr, W_enc, b_enc, lin0_W, lin0_b, mlp0_W1,
           mlp0_b1, mlp0_W2, mlp0_b2, ln0_g, ln0_b, lin1_W, lin1_b, mlp1_W1,
           mlp1_b1, mlp1_W2, mlp1_b2, ln1_g, ln1_b, W_ih, W_hh, b_ih, b_hh,
           W_head, b_head):
    B, T, N, F = x_seq.shape
    H = W_enc.shape[0]
    src = edge_index[0]
    dst = edge_index[1]

    r2 = lambda v: v.reshape(1, -1)

    e0p, e1p = _edge_embed(edge_attr, lin0_W.T, r2(lin0_b), lin1_W.T, r2(lin1_b))
    X, Xp = _encode(x_seq.reshape(T * N, F), W_enc.T, r2(b_enc))

    zeros_blk = jnp.zeros((((N + 127) // 128) * 128 // 16, H), F32)

    agg0 = _message(Xp, src, dst, e0p, zeros_blk, T, N, H)
    X1, X1p = _post(X, agg0, mlp0_W1.T, r2(mlp0_b1), mlp0_W2.T, r2(mlp0_b2),
                    r2(ln0_g), r2(ln0_b), emit_packed=True)
    agg1 = _message(X1p, src, dst, e1p, zeros_blk, T, N, H)
    X2, _ = _post(X1, agg1, mlp1_W1.T, r2(mlp1_b1), mlp1_W2.T, r2(mlp1_b2),
                  r2(ln1_g), r2(ln1_b), emit_packed=False)

    Np = ((N + 1023) // 1024) * 1024
    seq = jnp.pad(X2.reshape(T, N, H), ((0, 0), (0, Np - N), (0, 0)))
    hout = _gru_head(seq, W_ih.T, W_hh.T, r2(b_ih), r2(b_hh), W_head,
                     b_head.reshape(1, 1))
    return hout[:N, 0].reshape(1, N)


# speedup vs baseline: 1.8768x; 1.0571x over previous
import functools

import jax
import jax.numpy as jnp
from jax import lax
from jax.experimental import pallas as pl
from jax.experimental.pallas import tpu as pltpu
from jax.experimental.pallas import tpu_sc as plsc

F32 = jnp.float32
U32 = jnp.uint32


def _pack_bf16_pairs(y):
    yi = lax.bitcast_convert_type(y, U32)
    lsb = (yi >> 16) & jnp.uint32(1)
    r = (yi + jnp.uint32(0x7FFF) + lsb) >> 16
    half = y.shape[-1] // 2
    pk = r[:, :half] | (r[:, half:] << 16)
    return lax.bitcast_convert_type(pk, jnp.int32)


def _encode(x, w_t, b, block_rows=2000):
    M, K = x.shape
    Hout = w_t.shape[1]
    nb = M // block_rows

    def body(x_ref, w_ref, b_ref, o_ref, op_ref):
        y = jnp.dot(x_ref[...], w_ref[...], preferred_element_type=F32) + b_ref[...]
        o_ref[...] = y
        op_ref[...] = _pack_bf16_pairs(y)

    full = lambda i: (0, 0)
    return pl.pallas_call(
        body,
        grid=(nb,),
        in_specs=[
            pl.BlockSpec((block_rows, K), lambda i: (i, 0)),
            pl.BlockSpec((K, Hout), full),
            pl.BlockSpec((1, Hout), full),
        ],
        out_specs=[
            pl.BlockSpec((block_rows, Hout), lambda i: (i, 0)),
            pl.BlockSpec((block_rows, Hout // 2), lambda i: (i, 0)),
        ],
        out_shape=[
            jax.ShapeDtypeStruct((M, Hout), F32),
            jax.ShapeDtypeStruct((M, Hout // 2), jnp.int32),
        ],
    )(x, w_t, b)


def _edge_embed(attr, w0_t, b0, w1_t, b1, block_rows=2000):
    E, D = attr.shape
    H = w0_t.shape[1]
    nb = E // block_rows

    def body(a_ref, w0_ref, b0_ref, w1_ref, b1_ref, e0_ref, e1_ref):
        a = a_ref[...]
        e0_ref[...] = _pack_bf16_pairs(
            jnp.dot(a, w0_ref[...], preferred_element_type=F32) + b0_ref[...]
        )
        e1_ref[...] = _pack_bf16_pairs(
            jnp.dot(a, w1_ref[...], preferred_element_type=F32) + b1_ref[...]
        )

    full = lambda i: (0, 0)
    return pl.pallas_call(
        body,
        grid=(nb,),
        in_specs=[
            pl.BlockSpec((block_rows, D), lambda i: (i, 0)),
            pl.BlockSpec((D, H), full),
            pl.BlockSpec((1, H), full),
            pl.BlockSpec((D, H), full),
            pl.BlockSpec((1, H), full),
        ],
        out_specs=[
            pl.BlockSpec((block_rows, H // 2), lambda i: (i, 0)),
            pl.BlockSpec((block_rows, H // 2), lambda i: (i, 0)),
        ],
        out_shape=[
            jax.ShapeDtypeStruct((E, H // 2), jnp.int32),
            jax.ShapeDtypeStruct((E, H // 2), jnp.int32),
        ],
    )(attr, w0_t, b0, w1_t, b1)


def _post(x, agg, w1_t, b1, w2_t, b2, g, b, emit_packed, block_rows=2000):
    M, H = x.shape
    nb = M // block_rows
    full = lambda i: (0, 0)
    blk = lambda i: (i, 0)

    def body(x_ref, a_ref, w1_ref, b1_ref, w2_ref, b2_ref, g_ref, bb_ref,
             o_ref, *op_ref):
        x_ = x_ref[...]
        h = x_ + a_ref[...]
        y = jnp.maximum(
            jnp.dot(h, w1_ref[...], preferred_element_type=F32) + b1_ref[...], 0.0
        )
        y = jnp.dot(y, w2_ref[...], preferred_element_type=F32) + b2_ref[...]
        mu = jnp.mean(y, axis=-1, keepdims=True)
        var = jnp.mean((y - mu) ** 2, axis=-1, keepdims=True)
        z = (y - mu) * lax.rsqrt(var + 1e-5) * g_ref[...] + bb_ref[...]
        o = jnp.maximum(z, 0.0) + x_
        o_ref[...] = o
        if op_ref:
            op_ref[0][...] = _pack_bf16_pairs(o)

    out_specs = [pl.BlockSpec((block_rows, H), blk)]
    out_shape = [jax.ShapeDtypeStruct((M, H), F32)]
    if emit_packed:
        out_specs.append(pl.BlockSpec((block_rows, H // 2), blk))
        out_shape.append(jax.ShapeDtypeStruct((M, H // 2), jnp.int32))

    res = pl.pallas_call(
        body,
        grid=(nb,),
        in_specs=[pl.BlockSpec((block_rows, H), blk)] * 2
        + [
            pl.BlockSpec((H, H), full),
            pl.BlockSpec((1, H), full),
            pl.BlockSpec((H, H), full),
            pl.BlockSpec((1, H), full),
            pl.BlockSpec((1, H), full),
            pl.BlockSpec((1, H), full),
        ],
        out_specs=out_specs,
        out_shape=out_shape,
    )(x, agg, w1_t, b1, w2_t, b2, g, b)
    return res if emit_packed else (res[0], None)


def _gru_head(seq, wih_t, whh_t, bih, bhh, w_head, b_head, block_rows=1024):
    T, Np, H = seq.shape
    nb = Np // block_rows

    def body(s_ref, wih_ref, whh_ref, bih_ref, bhh_ref, wh_ref, bh_ref, o_ref):
        h = jnp.zeros((block_rows, H), F32)
        for t in range(T):
            xt = s_ref[t]
            gx = jnp.dot(xt, wih_ref[...], preferred_element_type=F32) + bih_ref[...]
            gh = jnp.dot(h, whh_ref[...], preferred_element_type=F32) + bhh_ref[...]
            r = jax.nn.sigmoid(gx[:, :H] + gh[:, :H])
            z = jax.nn.sigmoid(gx[:, H : 2 * H] + gh[:, H : 2 * H])
            n = jnp.tanh(gx[:, 2 * H :] + r * gh[:, 2 * H :])
            h = (1.0 - z) * n + z * h
        p = jnp.sum(h * wh_ref[...], axis=1, keepdims=True) + bh_ref[0, 0]
        o_ref[...] = jnp.broadcast_to(p, (block_rows, H))

    full = lambda i: (0, 0)
    return pl.pallas_call(
        body,
        grid=(nb,),
        in_specs=[
            pl.BlockSpec((T, block_rows, H), lambda i: (0, i, 0)),
            pl.BlockSpec((H, 3 * H), full),
            pl.BlockSpec((H, 3 * H), full),
            pl.BlockSpec((1, 3 * H), full),
            pl.BlockSpec((1, 3 * H), full),
            pl.BlockSpec((1, H), full),
            pl.BlockSpec((1, 1), full),
        ],
        out_specs=pl.BlockSpec((block_rows, H), lambda i: (i, 0)),
        out_shape=jax.ShapeDtypeStruct((Np, H), F32),
    )(seq, wih_t, whh_t, bih, bhh, w_head, b_head)


def _message(x_pk, src, dst, e_pk, zeros_blk, T, N, H):
    E = src.shape[0]
    HW = H // 2
    NSUB = 16
    NCORE = 2
    TP = T // NCORE
    CH = 80
    GCH = E // CH
    NCH = (GCH + NSUB - 1) // NSUB
    NCH = NCH + (NCH & 1)
    NP = ((N + 127) // 128) * 128
    NPS = NP // NSUB
    DUMP = NP - 8

    mesh = plsc.VectorSubcoreMesh(
        core_axis_name="c", subcore_axis_name="s", num_cores=NCORE, num_subcores=NSUB
    )

    @functools.partial(
        pl.kernel,
        out_type=jax.ShapeDtypeStruct((T * NP, H), F32),
        mesh=mesh,
        compiler_params=pltpu.CompilerParams(
            needs_layout_passes=False, use_tc_tiling_on_sc=False
        ),
        scratch_types=[
            [pltpu.VMEM((CH,), jnp.int32)] * 2,
            [pltpu.VMEM((CH,), jnp.int32)] * 2,
            [pltpu.VMEM((CH,), jnp.int32)] * 2,
            [pltpu.VMEM((CH, HW), jnp.int32)] * 2,
            [pltpu.VMEM((CH, HW), jnp.int32)] * 2,
            [pltpu.VMEM((CH, H), F32)] * 2,
            pltpu.VMEM_SHARED((NP, H), F32),
            [pltpu.SemaphoreType.DMA] * 2,
            [pltpu.SemaphoreType.DMA] * 2,
            [pltpu.SemaphoreType.DMA] * 2,
        ],
    )
    def msg(x_hbm, src_hbm, dst_hbm, e_hbm, z_hbm, out_hbm, src_v, dst_v, dsc_v,
            e_v, xr_v, xf_v, acc, fsem, gsem, ssem):
        c = lax.axis_index("c")
        s = lax.axis_index("s")

        def chunk_off(k):
            g = k * NSUB + s
            g = jnp.minimum(g, GCH - 1)
            return g * CH

        def fetch(k, b):
            off = chunk_off(k)
            pltpu.async_copy(src_hbm.at[pl.ds(off, CH)], src_v[b], fsem[b])
            pltpu.async_copy(dst_hbm.at[pl.ds(off, CH)], dst_v[b], fsem[b])
            pltpu.async_copy(e_hbm.at[pl.ds(off, CH)], e_v[b], fsem[b])

        def wait_fetch(b):
            pltpu.make_async_copy(src_hbm.at[pl.ds(0, CH)], src_v[b], fsem[b]).wait()
            pltpu.make_async_copy(dst_hbm.at[pl.ds(0, CH)], dst_v[b], fsem[b]).wait()
            pltpu.make_async_copy(e_hbm.at[pl.ds(0, CH)], e_v[b], fsem[b]).wait()

        def prep_idx(k, b, t):
            fake = (k * NSUB + s) >= GCH
            for g in range(CH // 16):
                sl = pl.ds(g * 16, 16)
                src_v[b][sl] = src_v[b][sl] + t * N
                dst_v[b][sl] = jnp.where(fake, DUMP, dst_v[b][sl])

        def gather(b):
            pltpu.async_copy(x_hbm.at[src_v[b]], xr_v[b], gsem[b])

        def wait_gather(b):
            pltpu.make_async_copy(x_hbm.at[src_v[b]], xr_v[b], gsem[b]).wait()

        def wait_scatter(b):
            pltpu.make_async_copy(xf_v[b], acc.at[dsc_v[b]], ssem[b]).wait()

        for j in range(TP):
            t = c * TP + j
            pltpu.sync_copy(z_hbm, acc.at[pl.ds(s * NPS, NPS)])
            plsc.subcore_barrier()

            fetch(0, 0)
            fetch(1, 1)
            wait_fetch(0)
            prep_idx(0, 0, t)
            gather(0)

            def step(ci2, _):
                for b in (0, 1):
                    ci = ci2 * 2 + b
                    nb = 1 - b

                    @pl.when(ci + 1 < NCH)
                    def _():
                        wait_fetch(nb)
                        prep_idx(ci + 1, nb, t)

                    @pl.when(ci >= 1)
                    def _():
                        wait_scatter(nb)

                    @pl.when(ci + 1 < NCH)
                    def _():
                        gather(nb)

                    wait_gather(b)

                    @plsc.parallel_loop(0, CH, unroll=4)
                    def row(i):
                        for g in range(HW // 16):
                            sl = pl.ds(g * 16, 16)
                            xb = plsc.bitcast(xr_v[b][i, sl], jnp.bfloat16)
                            eb = plsc.bitcast(e_v[b][i, sl], jnp.bfloat16)
                            m = jnp.maximum(xb + eb, jnp.zeros((32,), jnp.bfloat16))
                            w = plsc.bitcast(m, jnp.int32)
                            xf_v[b][i, pl.ds(g * 16, 16)] = plsc.bitcast(
                                w << 16, F32
                            )
                            xf_v[b][i, pl.ds(HW + g * 16, 16)] = plsc.bitcast(
                                w & jnp.int32(-65536), F32
                            )
                    for g in range(CH // 16):
                        sl = pl.ds(g * 16, 16)
                        dsc_v[b][sl] = dst_v[b][sl]
                    pltpu.async_copy(xf_v[b], acc.at[dsc_v[b]], ssem[b], add=True)

                    @pl.when(ci + 2 < NCH)
                    def _():
                        fetch(ci + 2, b)

                return 0

            lax.fori_loop(0, NCH // 2, step, 0)
            wait_scatter(1)
            plsc.subcore_barrier()
            pltpu.sync_copy(
                acc.at[pl.ds(s * NPS, NPS)],
                out_hbm.at[pl.ds(t * NP + s * NPS, NPS)],
            )

    NPpad = ((N + 127) // 128) * 128
    out = msg(x_pk, src, dst, e_pk, zeros_blk)
    return out.reshape(T, NPpad, H)[:, :N].reshape(T * N, H)


def kernel(x_seq, edge_index, edge_attr, W_enc, b_enc, lin0_W, lin0_b, mlp0_W1,
           mlp0_b1, mlp0_W2, mlp0_b2, ln0_g, ln0_b, lin1_W, lin1_b, mlp1_W1,
           mlp1_b1, mlp1_W2, mlp1_b2, ln1_g, ln1_b, W_ih, W_hh, b_ih, b_hh,
           W_head, b_head):
    B, T, N, F = x_seq.shape
    H = W_enc.shape[0]
    src = edge_index[0]
    dst = edge_index[1]

    r2 = lambda v: v.reshape(1, -1)

    e0p, e1p = _edge_embed(edge_attr, lin0_W.T, r2(lin0_b), lin1_W.T, r2(lin1_b))
    X, Xp = _encode(x_seq.reshape(T * N, F), W_enc.T, r2(b_enc))

    zeros_blk = jnp.zeros((((N + 127) // 128) * 128 // 16, H), F32)

    agg0 = _message(Xp, src, dst, e0p, zeros_blk, T, N, H)
    X1, X1p = _post(X, agg0, mlp0_W1.T, r2(mlp0_b1), mlp0_W2.T, r2(mlp0_b2),
                    r2(ln0_g), r2(ln0_b), emit_packed=True)
    agg1 = _message(X1p, src, dst, e1p, zeros_blk, T, N, H)
    X2, _ = _post(X1, agg1, mlp1_W1.T, r2(mlp1_b1), mlp1_W2.T, r2(mlp1_b2),
                  r2(ln1_g), r2(ln1_b), emit_packed=False)

    Np = ((N + 1023) // 1024) * 1024
    seq = jnp.pad(X2.reshape(T, N, H), ((0, 0), (0, Np - N), (0, 0)))
    hout = _gru_head(seq, W_ih.T, W_hh.T, r2(b_ih), r2(b_hh), W_head,
                     b_head.reshape(1, 1))
    return hout[:N, 0].reshape(1, N)


# speedup vs baseline: 1.8852x; 1.0044x over previous
import functools

import jax
import jax.numpy as jnp
from jax import lax
from jax.experimental import pallas as pl
from jax.experimental.pallas import tpu as pltpu
from jax.experimental.pallas import tpu_sc as plsc

F32 = jnp.float32
U32 = jnp.uint32


def _pack_bf16_pairs(y):
    yi = lax.bitcast_convert_type(y, U32)
    lsb = (yi >> 16) & jnp.uint32(1)
    r = (yi + jnp.uint32(0x7FFF) + lsb) >> 16
    half = y.shape[-1] // 2
    pk = r[:, :half] | (r[:, half:] << 16)
    return lax.bitcast_convert_type(pk, jnp.int32)


def _encode(x, w_t, b, block_rows=2000):
    M, K = x.shape
    Hout = w_t.shape[1]
    nb = M // block_rows

    def body(x_ref, w_ref, b_ref, o_ref, op_ref):
        y = jnp.dot(x_ref[...], w_ref[...], preferred_element_type=F32) + b_ref[...]
        o_ref[...] = y
        op_ref[...] = _pack_bf16_pairs(y)

    full = lambda i: (0, 0)
    return pl.pallas_call(
        body,
        grid=(nb,),
        in_specs=[
            pl.BlockSpec((block_rows, K), lambda i: (i, 0)),
            pl.BlockSpec((K, Hout), full),
            pl.BlockSpec((1, Hout), full),
        ],
        out_specs=[
            pl.BlockSpec((block_rows, Hout), lambda i: (i, 0)),
            pl.BlockSpec((block_rows, Hout // 2), lambda i: (i, 0)),
        ],
        out_shape=[
            jax.ShapeDtypeStruct((M, Hout), F32),
            jax.ShapeDtypeStruct((M, Hout // 2), jnp.int32),
        ],
    )(x, w_t, b)


def _edge_embed(attr, w0_t, b0, w1_t, b1, block_rows=2000):
    E, D = attr.shape
    H = w0_t.shape[1]
    nb = E // block_rows

    def body(a_ref, w0_ref, b0_ref, w1_ref, b1_ref, e0_ref, e1_ref):
        a = a_ref[...]
        e0_ref[...] = _pack_bf16_pairs(
            jnp.dot(a, w0_ref[...], preferred_element_type=F32) + b0_ref[...]
        )
        e1_ref[...] = _pack_bf16_pairs(
            jnp.dot(a, w1_ref[...], preferred_element_type=F32) + b1_ref[...]
        )

    full = lambda i: (0, 0)
    return pl.pallas_call(
        body,
        grid=(nb,),
        in_specs=[
            pl.BlockSpec((block_rows, D), lambda i: (i, 0)),
            pl.BlockSpec((D, H), full),
            pl.BlockSpec((1, H), full),
            pl.BlockSpec((D, H), full),
            pl.BlockSpec((1, H), full),
        ],
        out_specs=[
            pl.BlockSpec((block_rows, H // 2), lambda i: (i, 0)),
            pl.BlockSpec((block_rows, H // 2), lambda i: (i, 0)),
        ],
        out_shape=[
            jax.ShapeDtypeStruct((E, H // 2), jnp.int32),
            jax.ShapeDtypeStruct((E, H // 2), jnp.int32),
        ],
    )(attr, w0_t, b0, w1_t, b1)


def _post(x, agg, w1_t, b1, w2_t, b2, g, b, emit_packed, block_rows=2000):
    M, H = x.shape
    nb = M // block_rows
    full = lambda i: (0, 0)
    blk = lambda i: (i, 0)

    def body(x_ref, a_ref, w1_ref, b1_ref, w2_ref, b2_ref, g_ref, bb_ref,
             o_ref, *op_ref):
        x_ = x_ref[...]
        h = x_ + a_ref[...]
        y = jnp.maximum(
            jnp.dot(h, w1_ref[...], preferred_element_type=F32) + b1_ref[...], 0.0
        )
        y = jnp.dot(y, w2_ref[...], preferred_element_type=F32) + b2_ref[...]
        mu = jnp.mean(y, axis=-1, keepdims=True)
        var = jnp.mean((y - mu) ** 2, axis=-1, keepdims=True)
        z = (y - mu) * lax.rsqrt(var + 1e-5) * g_ref[...] + bb_ref[...]
        o = jnp.maximum(z, 0.0) + x_
        o_ref[...] = o
        if op_ref:
            op_ref[0][...] = _pack_bf16_pairs(o)

    out_specs = [pl.BlockSpec((block_rows, H), blk)]
    out_shape = [jax.ShapeDtypeStruct((M, H), F32)]
    if emit_packed:
        out_specs.append(pl.BlockSpec((block_rows, H // 2), blk))
        out_shape.append(jax.ShapeDtypeStruct((M, H // 2), jnp.int32))

    res = pl.pallas_call(
        body,
        grid=(nb,),
        in_specs=[pl.BlockSpec((block_rows, H), blk)] * 2
        + [
            pl.BlockSpec((H, H), full),
            pl.BlockSpec((1, H), full),
            pl.BlockSpec((H, H), full),
            pl.BlockSpec((1, H), full),
            pl.BlockSpec((1, H), full),
            pl.BlockSpec((1, H), full),
        ],
        out_specs=out_specs,
        out_shape=out_shape,
    )(x, agg, w1_t, b1, w2_t, b2, g, b)
    return res if emit_packed else (res[0], None)


def _gru_head(seq, wih_t, whh_t, bih, bhh, w_head, b_head, block_rows=1024):
    T, Np, H = seq.shape
    nb = Np // block_rows

    def body(s_ref, wih_ref, whh_ref, bih_ref, bhh_ref, wh_ref, bh_ref, o_ref):
        h = jnp.zeros((block_rows, H), F32)
        for t in range(T):
            xt = s_ref[t]
            gx = jnp.dot(xt, wih_ref[...], preferred_element_type=F32) + bih_ref[...]
            gh = jnp.dot(h, whh_ref[...], preferred_element_type=F32) + bhh_ref[...]
            r = jax.nn.sigmoid(gx[:, :H] + gh[:, :H])
            z = jax.nn.sigmoid(gx[:, H : 2 * H] + gh[:, H : 2 * H])
            n = jnp.tanh(gx[:, 2 * H :] + r * gh[:, 2 * H :])
            h = (1.0 - z) * n + z * h
        p = jnp.sum(h * wh_ref[...], axis=1, keepdims=True) + bh_ref[0, 0]
        o_ref[...] = jnp.broadcast_to(p, (block_rows, H))

    full = lambda i: (0, 0)
    return pl.pallas_call(
        body,
        grid=(nb,),
        in_specs=[
            pl.BlockSpec((T, block_rows, H), lambda i: (0, i, 0)),
            pl.BlockSpec((H, 3 * H), full),
            pl.BlockSpec((H, 3 * H), full),
            pl.BlockSpec((1, 3 * H), full),
            pl.BlockSpec((1, 3 * H), full),
            pl.BlockSpec((1, H), full),
            pl.BlockSpec((1, 1), full),
        ],
        out_specs=pl.BlockSpec((block_rows, H), lambda i: (i, 0)),
        out_shape=jax.ShapeDtypeStruct((Np, H), F32),
    )(seq, wih_t, whh_t, bih, bhh, w_head, b_head)


def _message(x_pk, src, dst, e_pk, zeros_blk, T, N, H):
    E = src.shape[0]
    HW = H // 2
    NSUB = 16
    NCORE = 2
    TP = T // NCORE
    CH = 80
    GCH = E // CH
    NCH = (GCH + NSUB - 1) // NSUB
    NCH = NCH + (NCH & 1)
    NP = ((N + 127) // 128) * 128
    NPS = NP // NSUB
    DUMP = NP - 8

    mesh = plsc.VectorSubcoreMesh(
        core_axis_name="c", subcore_axis_name="s", num_cores=NCORE, num_subcores=NSUB
    )

    @functools.partial(
        pl.kernel,
        out_type=jax.ShapeDtypeStruct((T * NP, H), F32),
        mesh=mesh,
        compiler_params=pltpu.CompilerParams(
            needs_layout_passes=False, use_tc_tiling_on_sc=False
        ),
        scratch_types=[
            [pltpu.VMEM((CH,), jnp.int32)] * 2,
            [pltpu.VMEM((CH,), jnp.int32)] * 2,
            [pltpu.VMEM((CH,), jnp.int32)] * 2,
            [pltpu.VMEM((CH, HW), jnp.int32)] * 2,
            [pltpu.VMEM((CH, HW), jnp.int32)] * 2,
            [pltpu.VMEM((CH, H), F32)] * 2,
            pltpu.VMEM_SHARED((NP, H), F32),
            [pltpu.SemaphoreType.DMA] * 2,
            [pltpu.SemaphoreType.DMA] * 2,
            [pltpu.SemaphoreType.DMA] * 2,
        ],
    )
    def msg(x_hbm, src_hbm, dst_hbm, e_hbm, z_hbm, out_hbm, src_v, dst_v, dsc_v,
            e_v, xr_v, xf_v, acc, fsem, gsem, ssem):
        c = lax.axis_index("c")
        s = lax.axis_index("s")

        def chunk_off(k):
            g = k * NSUB + s
            g = jnp.minimum(g, GCH - 1)
            return g * CH

        def fetch(k, b):
            off = chunk_off(k)
            pltpu.async_copy(src_hbm.at[pl.ds(off, CH)], src_v[b], fsem[b])
            pltpu.async_copy(dst_hbm.at[pl.ds(off, CH)], dst_v[b], fsem[b])
            pltpu.async_copy(e_hbm.at[pl.ds(off, CH)], e_v[b], fsem[b])

        def wait_fetch(b):
            pltpu.make_async_copy(src_hbm.at[pl.ds(0, CH)], src_v[b], fsem[b]).wait()
            pltpu.make_async_copy(dst_hbm.at[pl.ds(0, CH)], dst_v[b], fsem[b]).wait()
            pltpu.make_async_copy(e_hbm.at[pl.ds(0, CH)], e_v[b], fsem[b]).wait()

        def prep_idx(k, b, t):
            fake = (k * NSUB + s) >= GCH
            for g in range(CH // 16):
                sl = pl.ds(g * 16, 16)
                src_v[b][sl] = src_v[b][sl] + t * N
                dst_v[b][sl] = jnp.where(fake, DUMP, dst_v[b][sl])

        def gather(b):
            pltpu.async_copy(x_hbm.at[src_v[b]], xr_v[b], gsem[b])

        def wait_gather(b):
            pltpu.make_async_copy(x_hbm.at[src_v[b]], xr_v[b], gsem[b]).wait()

        def wait_scatter(b):
            pltpu.make_async_copy(xf_v[b], acc.at[dsc_v[b]], ssem[b]).wait()

        for j in range(TP):
            t = c * TP + j
            pltpu.sync_copy(z_hbm, acc.at[pl.ds(s * NPS, NPS)])
            plsc.subcore_barrier()

            fetch(0, 0)
            fetch(1, 1)
            wait_fetch(0)
            prep_idx(0, 0, t)
            gather(0)

            def step(ci2, _):
                for b in (0, 1):
                    ci = ci2 * 2 + b
                    nb = 1 - b

                    @pl.when(ci + 1 < NCH)
                    def _():
                        wait_fetch(nb)
                        prep_idx(ci + 1, nb, t)

                    @pl.when(ci >= 1)
                    def _():
                        wait_scatter(nb)

                    @pl.when(ci + 1 < NCH)
                    def _():
                        gather(nb)

                    wait_gather(b)

                    @plsc.parallel_loop(0, CH, unroll=8)
                    def row(i):
                        for g in range(HW // 16):
                            sl = pl.ds(g * 16, 16)
                            xb = plsc.bitcast(xr_v[b][i, sl], jnp.bfloat16)
                            eb = plsc.bitcast(e_v[b][i, sl], jnp.bfloat16)
                            m = jnp.maximum(xb + eb, jnp.zeros((32,), jnp.bfloat16))
                            w = plsc.bitcast(m, jnp.int32)
                            xf_v[b][i, pl.ds(g * 16, 16)] = plsc.bitcast(
                                w << 16, F32
                            )
                            xf_v[b][i, pl.ds(HW + g * 16, 16)] = plsc.bitcast(
                                w & jnp.int32(-65536), F32
                            )
                    for g in range(CH // 16):
                        sl = pl.ds(g * 16, 16)
                        dsc_v[b][sl] = dst_v[b][sl]
                    pltpu.async_copy(xf_v[b], acc.at[dsc_v[b]], ssem[b], add=True)

                    @pl.when(ci + 2 < NCH)
                    def _():
                        fetch(ci + 2, b)

                return 0

            lax.fori_loop(0, NCH // 2, step, 0)
            wait_scatter(1)
            plsc.subcore_barrier()
            pltpu.sync_copy(
                acc.at[pl.ds(s * NPS, NPS)],
                out_hbm.at[pl.ds(t * NP + s * NPS, NPS)],
            )

    NPpad = ((N + 127) // 128) * 128
    out = msg(x_pk, src, dst, e_pk, zeros_blk)
    return out.reshape(T, NPpad, H)[:, :N].reshape(T * N, H)


def kernel(x_seq, edge_index, edge_attr, W_enc, b_enc, lin0_W, lin0_b, mlp0_W1,
           mlp0_b1, mlp0_W2, mlp0_b2, ln0_g, ln0_b, lin1_W, lin1_b, mlp1_W1,
           mlp1_b1, mlp1_W2, mlp1_b2, ln1_g, ln1_b, W_ih, W_hh, b_ih, b_hh,
           W_head, b_head):
    B, T, N, F = x_seq.shape
    H = W_enc.shape[0]
    src = edge_index[0]
    dst = edge_index[1]

    r2 = lambda v: v.reshape(1, -1)

    e0p, e1p = _edge_embed(edge_attr, lin0_W.T, r2(lin0_b), lin1_W.T, r2(lin1_b))
    X, Xp = _encode(x_seq.reshape(T * N, F), W_enc.T, r2(b_enc))

    zeros_blk = jnp.zeros((((N + 127) // 128) * 128 // 16, H), F32)

    agg0 = _message(Xp, src, dst, e0p, zeros_blk, T, N, H)
    X1, X1p = _post(X, agg0, mlp0_W1.T, r2(mlp0_b1), mlp0_W2.T, r2(mlp0_b2),
                    r2(ln0_g), r2(ln0_b), emit_packed=True)
    agg1 = _message(X1p, src, dst, e1p, zeros_blk, T, N, H)
    X2, _ = _post(X1, agg1, mlp1_W1.T, r2(mlp1_b1), mlp1_W2.T, r2(mlp1_b2),
                  r2(ln1_g), r2(ln1_b), emit_packed=False)

    Np = ((N + 1023) // 1024) * 1024
    seq = jnp.pad(X2.reshape(T, N, H), ((0, 0), (0, Np - N), (0, 0)))
    hout = _gru_head(seq, W_ih.T, W_hh.T, r2(b_ih), r2(b_hh), W_head,
                     b_head.reshape(1, 1))
    return hout[:N, 0].reshape(1, N)


# speedup vs baseline: 1.9101x; 1.0132x over previous
import functools

import jax
import jax.numpy as jnp
from jax import lax
from jax.experimental import pallas as pl
from jax.experimental.pallas import tpu as pltpu
from jax.experimental.pallas import tpu_sc as plsc

F32 = jnp.float32
U32 = jnp.uint32


def _pack_bf16_pairs(y):
    yi = lax.bitcast_convert_type(y, U32)
    lsb = (yi >> 16) & jnp.uint32(1)
    r = (yi + jnp.uint32(0x7FFF) + lsb) >> 16
    half = y.shape[-1] // 2
    pk = r[:, :half] | (r[:, half:] << 16)
    return lax.bitcast_convert_type(pk, jnp.int32)


def _encode(x, w_t, b, block_rows=2000):
    M, K = x.shape
    Hout = w_t.shape[1]
    nb = M // block_rows

    def body(x_ref, w_ref, b_ref, o_ref, op_ref):
        y = jnp.dot(x_ref[...], w_ref[...], preferred_element_type=F32) + b_ref[...]
        o_ref[...] = y
        op_ref[...] = _pack_bf16_pairs(y)

    full = lambda i: (0, 0)
    return pl.pallas_call(
        body,
        grid=(nb,),
        in_specs=[
            pl.BlockSpec((block_rows, K), lambda i: (i, 0)),
            pl.BlockSpec((K, Hout), full),
            pl.BlockSpec((1, Hout), full),
        ],
        out_specs=[
            pl.BlockSpec((block_rows, Hout), lambda i: (i, 0)),
            pl.BlockSpec((block_rows, Hout // 2), lambda i: (i, 0)),
        ],
        out_shape=[
            jax.ShapeDtypeStruct((M, Hout), F32),
            jax.ShapeDtypeStruct((M, Hout // 2), jnp.int32),
        ],
    )(x, w_t, b)


def _edge_embed(attr, w0_t, b0, w1_t, b1, block_rows=2000):
    E, D = attr.shape
    H = w0_t.shape[1]
    nb = E // block_rows

    def body(a_ref, w0_ref, b0_ref, w1_ref, b1_ref, e0_ref, e1_ref):
        a = a_ref[...]
        e0_ref[...] = _pack_bf16_pairs(
            jnp.dot(a, w0_ref[...], preferred_element_type=F32) + b0_ref[...]
        )
        e1_ref[...] = _pack_bf16_pairs(
            jnp.dot(a, w1_ref[...], preferred_element_type=F32) + b1_ref[...]
        )

    full = lambda i: (0, 0)
    return pl.pallas_call(
        body,
        grid=(nb,),
        in_specs=[
            pl.BlockSpec((block_rows, D), lambda i: (i, 0)),
            pl.BlockSpec((D, H), full),
            pl.BlockSpec((1, H), full),
            pl.BlockSpec((D, H), full),
            pl.BlockSpec((1, H), full),
        ],
        out_specs=[
            pl.BlockSpec((block_rows, H // 2), lambda i: (i, 0)),
            pl.BlockSpec((block_rows, H // 2), lambda i: (i, 0)),
        ],
        out_shape=[
            jax.ShapeDtypeStruct((E, H // 2), jnp.int32),
            jax.ShapeDtypeStruct((E, H // 2), jnp.int32),
        ],
    )(attr, w0_t, b0, w1_t, b1)


def _post(x, agg, w1_t, b1, w2_t, b2, g, b, emit_packed, block_rows=2000):
    M, H = x.shape
    nb = M // block_rows
    full = lambda i: (0, 0)
    blk = lambda i: (i, 0)

    def body(x_ref, a_ref, w1_ref, b1_ref, w2_ref, b2_ref, g_ref, bb_ref,
             o_ref, *op_ref):
        x_ = x_ref[...]
        h = x_ + a_ref[...]
        y = jnp.maximum(
            jnp.dot(h, w1_ref[...], preferred_element_type=F32) + b1_ref[...], 0.0
        )
        y = jnp.dot(y, w2_ref[...], preferred_element_type=F32) + b2_ref[...]
        mu = jnp.mean(y, axis=-1, keepdims=True)
        var = jnp.mean((y - mu) ** 2, axis=-1, keepdims=True)
        z = (y - mu) * lax.rsqrt(var + 1e-5) * g_ref[...] + bb_ref[...]
        o = jnp.maximum(z, 0.0) + x_
        o_ref[...] = o
        if op_ref:
            op_ref[0][...] = _pack_bf16_pairs(o)

    out_specs = [pl.BlockSpec((block_rows, H), blk)]
    out_shape = [jax.ShapeDtypeStruct((M, H), F32)]
    if emit_packed:
        out_specs.append(pl.BlockSpec((block_rows, H // 2), blk))
        out_shape.append(jax.ShapeDtypeStruct((M, H // 2), jnp.int32))

    res = pl.pallas_call(
        body,
        grid=(nb,),
        in_specs=[pl.BlockSpec((block_rows, H), blk)] * 2
        + [
            pl.BlockSpec((H, H), full),
            pl.BlockSpec((1, H), full),
            pl.BlockSpec((H, H), full),
            pl.BlockSpec((1, H), full),
            pl.BlockSpec((1, H), full),
            pl.BlockSpec((1, H), full),
        ],
        out_specs=out_specs,
        out_shape=out_shape,
    )(x, agg, w1_t, b1, w2_t, b2, g, b)
    return res if emit_packed else (res[0], None)


def _gru_head(seq, wih_t, whh_t, bih, bhh, w_head, b_head, block_rows=1024):
    T, Np, H = seq.shape
    nb = Np // block_rows

    def body(s_ref, wih_ref, whh_ref, bih_ref, bhh_ref, wh_ref, bh_ref, o_ref):
        h = jnp.zeros((block_rows, H), F32)
        for t in range(T):
            xt = s_ref[t]
            gx = jnp.dot(xt, wih_ref[...], preferred_element_type=F32) + bih_ref[...]
            gh = jnp.dot(h, whh_ref[...], preferred_element_type=F32) + bhh_ref[...]
            r = jax.nn.sigmoid(gx[:, :H] + gh[:, :H])
            z = jax.nn.sigmoid(gx[:, H : 2 * H] + gh[:, H : 2 * H])
            n = jnp.tanh(gx[:, 2 * H :] + r * gh[:, 2 * H :])
            h = (1.0 - z) * n + z * h
        p = jnp.sum(h * wh_ref[...], axis=1, keepdims=True) + bh_ref[0, 0]
        o_ref[...] = jnp.broadcast_to(p, (block_rows, H))

    full = lambda i: (0, 0)
    return pl.pallas_call(
        body,
        grid=(nb,),
        in_specs=[
            pl.BlockSpec((T, block_rows, H), lambda i: (0, i, 0)),
            pl.BlockSpec((H, 3 * H), full),
            pl.BlockSpec((H, 3 * H), full),
            pl.BlockSpec((1, 3 * H), full),
            pl.BlockSpec((1, 3 * H), full),
            pl.BlockSpec((1, H), full),
            pl.BlockSpec((1, 1), full),
        ],
        out_specs=pl.BlockSpec((block_rows, H), lambda i: (i, 0)),
        out_shape=jax.ShapeDtypeStruct((Np, H), F32),
    )(seq, wih_t, whh_t, bih, bhh, w_head, b_head)


def _message(x_pk, src, dst, e_pk, zeros_blk, t_base, N, H):
    E = src.shape[0]
    HW = H // 2
    NSUB = 16
    NCORE = 2
    CH = 80
    GCH = E // CH
    NCH = (GCH + NSUB - 1) // NSUB
    NCH = NCH + (NCH & 1)
    NP = ((N + 127) // 128) * 128
    NPS = NP // NSUB
    DUMP = NP - 8

    mesh = plsc.VectorSubcoreMesh(
        core_axis_name="c", subcore_axis_name="s", num_cores=NCORE, num_subcores=NSUB
    )

    @functools.partial(
        pl.kernel,
        out_type=jax.ShapeDtypeStruct((NCORE * NP, H), F32),
        mesh=mesh,
        compiler_params=pltpu.CompilerParams(
            needs_layout_passes=False, use_tc_tiling_on_sc=False
        ),
        scratch_types=[
            [pltpu.VMEM((CH,), jnp.int32)] * 2,
            [pltpu.VMEM((CH,), jnp.int32)] * 2,
            [pltpu.VMEM((CH,), jnp.int32)] * 2,
            [pltpu.VMEM((CH, HW), jnp.int32)] * 2,
            [pltpu.VMEM((CH, HW), jnp.int32)] * 2,
            [pltpu.VMEM((CH, H), F32)] * 2,
            pltpu.VMEM_SHARED((NP, H), F32),
            [pltpu.SemaphoreType.DMA] * 2,
            [pltpu.SemaphoreType.DMA] * 2,
            [pltpu.SemaphoreType.DMA] * 2,
        ],
    )
    def msg(x_hbm, src_hbm, dst_hbm, e_hbm, z_hbm, out_hbm, src_v, dst_v, dsc_v,
            e_v, xr_v, xf_v, acc, fsem, gsem, ssem):
        c = lax.axis_index("c")
        s = lax.axis_index("s")

        def chunk_off(k):
            g = k * NSUB + s
            g = jnp.minimum(g, GCH - 1)
            return g * CH

        def fetch(k, b):
            off = chunk_off(k)
            pltpu.async_copy(src_hbm.at[pl.ds(off, CH)], src_v[b], fsem[b])
            pltpu.async_copy(dst_hbm.at[pl.ds(off, CH)], dst_v[b], fsem[b])
            pltpu.async_copy(e_hbm.at[pl.ds(off, CH)], e_v[b], fsem[b])

        def wait_fetch(b):
            pltpu.make_async_copy(src_hbm.at[pl.ds(0, CH)], src_v[b], fsem[b]).wait()
            pltpu.make_async_copy(dst_hbm.at[pl.ds(0, CH)], dst_v[b], fsem[b]).wait()
            pltpu.make_async_copy(e_hbm.at[pl.ds(0, CH)], e_v[b], fsem[b]).wait()

        def prep_idx(k, b, t):
            fake = (k * NSUB + s) >= GCH
            for g in range(CH // 16):
                sl = pl.ds(g * 16, 16)
                src_v[b][sl] = src_v[b][sl] + t * N
                dst_v[b][sl] = jnp.where(fake, DUMP, dst_v[b][sl])

        def gather(b):
            pltpu.async_copy(x_hbm.at[src_v[b]], xr_v[b], gsem[b])

        def wait_gather(b):
            pltpu.make_async_copy(x_hbm.at[src_v[b]], xr_v[b], gsem[b]).wait()

        def wait_scatter(b):
            pltpu.make_async_copy(xf_v[b], acc.at[dsc_v[b]], ssem[b]).wait()

        if True:
            t = t_base + c
            pltpu.sync_copy(z_hbm, acc.at[pl.ds(s * NPS, NPS)])
            plsc.subcore_barrier()

            fetch(0, 0)
            fetch(1, 1)
            wait_fetch(0)
            prep_idx(0, 0, t)
            gather(0)

            def step(ci2, _):
                for b in (0, 1):
                    ci = ci2 * 2 + b
                    nb = 1 - b

                    @pl.when(ci + 1 < NCH)
                    def _():
                        wait_fetch(nb)
                        prep_idx(ci + 1, nb, t)

                    @pl.when(ci >= 1)
                    def _():
                        wait_scatter(nb)

                    @pl.when(ci + 1 < NCH)
                    def _():
                        gather(nb)

                    wait_gather(b)

                    @plsc.parallel_loop(0, CH, unroll=8)
                    def row(i):
                        for g in range(HW // 16):
                            sl = pl.ds(g * 16, 16)
                            xb = plsc.bitcast(xr_v[b][i, sl], jnp.bfloat16)
                            eb = plsc.bitcast(e_v[b][i, sl], jnp.bfloat16)
                            m = jnp.maximum(xb + eb, jnp.zeros((32,), jnp.bfloat16))
                            w = plsc.bitcast(m, jnp.int32)
                            xf_v[b][i, pl.ds(g * 16, 16)] = plsc.bitcast(
                                w << 16, F32
                            )
                            xf_v[b][i, pl.ds(HW + g * 16, 16)] = plsc.bitcast(
                                w & jnp.int32(-65536), F32
                            )
                    for g in range(CH // 16):
                        sl = pl.ds(g * 16, 16)
                        dsc_v[b][sl] = dst_v[b][sl]
                    pltpu.async_copy(xf_v[b], acc.at[dsc_v[b]], ssem[b], add=True)

                    @pl.when(ci + 2 < NCH)
                    def _():
                        fetch(ci + 2, b)

                return 0

            lax.fori_loop(0, NCH // 2, step, 0)
            wait_scatter(1)
            plsc.subcore_barrier()
            pltpu.sync_copy(
                acc.at[pl.ds(s * NPS, NPS)],
                out_hbm.at[pl.ds(c * NP + s * NPS, NPS)],
            )

    NPpad = ((N + 127) // 128) * 128
    out = msg(x_pk, src, dst, e_pk, zeros_blk)
    return out.reshape(2, NPpad, H)[:, :N].reshape(2 * N, H)


def kernel(x_seq, edge_index, edge_attr, W_enc, b_enc, lin0_W, lin0_b, mlp0_W1,
           mlp0_b1, mlp0_W2, mlp0_b2, ln0_g, ln0_b, lin1_W, lin1_b, mlp1_W1,
           mlp1_b1, mlp1_W2, mlp1_b2, ln1_g, ln1_b, W_ih, W_hh, b_ih, b_hh,
           W_head, b_head):
    B, T, N, F = x_seq.shape
    H = W_enc.shape[0]
    src = edge_index[0]
    dst = edge_index[1]

    r2 = lambda v: v.reshape(1, -1)

    e0p, e1p = _edge_embed(edge_attr, lin0_W.T, r2(lin0_b), lin1_W.T, r2(lin1_b))
    X, Xp = _encode(x_seq.reshape(T * N, F), W_enc.T, r2(b_enc))

    zeros_blk = jnp.zeros((((N + 127) // 128) * 128 // 16, H), F32)

    agg0_a = _message(Xp, src, dst, e0p, zeros_blk, 0, N, H)
    agg0_b = _message(Xp, src, dst, e0p, zeros_blk, 2, N, H)
    post0 = lambda xs, ag: _post(
        xs, ag, mlp0_W1.T, r2(mlp0_b1), mlp0_W2.T, r2(mlp0_b2), r2(ln0_g),
        r2(ln0_b), emit_packed=True,
    )
    X1a, X1pa = post0(X[: 2 * N], agg0_a)
    X1b, X1pb = post0(X[2 * N :], agg0_b)
    agg1_a = _message(X1pa, src, dst, e1p, zeros_blk, 0, N, H)
    agg1_b = _message(X1pb, src, dst, e1p, zeros_blk, 0, N, H)
    post1 = lambda xs, ag: _post(
        xs, ag, mlp1_W1.T, r2(mlp1_b1), mlp1_W2.T, r2(mlp1_b2), r2(ln1_g),
        r2(ln1_b), emit_packed=False,
    )[0]
    X2 = jnp.concatenate([post1(X1a, agg1_a), post1(X1b, agg1_b)], axis=0)

    seq = X2.reshape(T, N, H)
    hout = _gru_head(seq, W_ih.T, W_hh.T, r2(b_ih), r2(b_hh), W_head,
                     b_head.reshape(1, 1), block_rows=1000)
    return hout[:, 0].reshape(1, N)


# speedup vs baseline: 1.9320x; 1.0114x over previous
import functools

import jax
import jax.numpy as jnp
from jax import lax
from jax.experimental import pallas as pl
from jax.experimental.pallas import tpu as pltpu
from jax.experimental.pallas import tpu_sc as plsc

F32 = jnp.float32
U32 = jnp.uint32


def _pack_bf16_pairs(y):
    yi = lax.bitcast_convert_type(y, U32)
    lsb = (yi >> 16) & jnp.uint32(1)
    r = (yi + jnp.uint32(0x7FFF) + lsb) >> 16
    half = y.shape[-1] // 2
    pk = r[:, :half] | (r[:, half:] << 16)
    return lax.bitcast_convert_type(pk, jnp.int32)


def _encode(x, w_t, b, block_rows=2000):
    M, K = x.shape
    Hout = w_t.shape[1]
    nb = M // block_rows

    def body(x_ref, w_ref, b_ref, o_ref, op_ref):
        y = jnp.dot(x_ref[...], w_ref[...], preferred_element_type=F32) + b_ref[...]
        o_ref[...] = y
        op_ref[...] = _pack_bf16_pairs(y)

    full = lambda i: (0, 0)
    return pl.pallas_call(
        body,
        grid=(nb,),
        in_specs=[
            pl.BlockSpec((block_rows, K), lambda i: (i, 0)),
            pl.BlockSpec((K, Hout), full),
            pl.BlockSpec((1, Hout), full),
        ],
        out_specs=[
            pl.BlockSpec((block_rows, Hout), lambda i: (i, 0)),
            pl.BlockSpec((block_rows, Hout // 2), lambda i: (i, 0)),
        ],
        out_shape=[
            jax.ShapeDtypeStruct((M, Hout), F32),
            jax.ShapeDtypeStruct((M, Hout // 2), jnp.int32),
        ],
    )(x, w_t, b)


def _edge_embed(attr, w0_t, b0, w1_t, b1, block_rows=2000):
    E, D = attr.shape
    H = w0_t.shape[1]
    nb = E // block_rows

    def body(a_ref, w0_ref, b0_ref, w1_ref, b1_ref, e0_ref, e1_ref):
        a = a_ref[...]
        e0_ref[...] = _pack_bf16_pairs(
            jnp.dot(a, w0_ref[...], preferred_element_type=F32) + b0_ref[...]
        )
        e1_ref[...] = _pack_bf16_pairs(
            jnp.dot(a, w1_ref[...], preferred_element_type=F32) + b1_ref[...]
        )

    full = lambda i: (0, 0)
    return pl.pallas_call(
        body,
        grid=(nb,),
        in_specs=[
            pl.BlockSpec((block_rows, D), lambda i: (i, 0)),
            pl.BlockSpec((D, H), full),
            pl.BlockSpec((1, H), full),
            pl.BlockSpec((D, H), full),
            pl.BlockSpec((1, H), full),
        ],
        out_specs=[
            pl.BlockSpec((block_rows, H // 2), lambda i: (i, 0)),
            pl.BlockSpec((block_rows, H // 2), lambda i: (i, 0)),
        ],
        out_shape=[
            jax.ShapeDtypeStruct((E, H // 2), jnp.int32),
            jax.ShapeDtypeStruct((E, H // 2), jnp.int32),
        ],
    )(attr, w0_t, b0, w1_t, b1)


def _post(x, agg3, w1_t, b1, w2_t, b2, g, b, emit_packed, x_ofs=0,
          block_rows=1000):
    _, NP, H = agg3.shape
    M = 2 * (block_rows * (NP // block_rows))
    npb = NP // block_rows
    nb = M // block_rows
    full = lambda i: (0, 0)
    blk = lambda i: (i, 0)

    def body(x_ref, a_ref, w1_ref, b1_ref, w2_ref, b2_ref, g_ref, bb_ref,
             o_ref, *op_ref):
        x_ = x_ref[...]
        h = x_ + a_ref[0]
        y = jnp.maximum(
            jnp.dot(h, w1_ref[...], preferred_element_type=F32) + b1_ref[...], 0.0
        )
        y = jnp.dot(y, w2_ref[...], preferred_element_type=F32) + b2_ref[...]
        mu = jnp.mean(y, axis=-1, keepdims=True)
        var = jnp.mean((y - mu) ** 2, axis=-1, keepdims=True)
        z = (y - mu) * lax.rsqrt(var + 1e-5) * g_ref[...] + bb_ref[...]
        o = jnp.maximum(z, 0.0) + x_
        o_ref[...] = o
        if op_ref:
            op_ref[0][...] = _pack_bf16_pairs(o)

    out_specs = [pl.BlockSpec((block_rows, H), blk)]
    out_shape = [jax.ShapeDtypeStruct((M, H), F32)]
    if emit_packed:
        out_specs.append(pl.BlockSpec((block_rows, H // 2), blk))
        out_shape.append(jax.ShapeDtypeStruct((M, H // 2), jnp.int32))

    res = pl.pallas_call(
        body,
        grid=(nb,),
        in_specs=[
            pl.BlockSpec((block_rows, H), lambda i: (x_ofs + i, 0)),
            pl.BlockSpec((1, block_rows, H), lambda i: (i // npb, i % npb, 0)),
        ]
        + [
            pl.BlockSpec((H, H), full),
            pl.BlockSpec((1, H), full),
            pl.BlockSpec((H, H), full),
            pl.BlockSpec((1, H), full),
            pl.BlockSpec((1, H), full),
            pl.BlockSpec((1, H), full),
        ],
        out_specs=out_specs,
        out_shape=out_shape,
    )(x, agg3, w1_t, b1, w2_t, b2, g, b)
    return res if emit_packed else (res[0], None)


def _gru_head(seq, wih_t, whh_t, bih, bhh, w_head, b_head, block_rows=1024):
    T, Np, H = seq.shape
    nb = Np // block_rows

    def body(s_ref, wih_ref, whh_ref, bih_ref, bhh_ref, wh_ref, bh_ref, o_ref):
        h = jnp.zeros((block_rows, H), F32)
        for t in range(T):
            xt = s_ref[t]
            gx = jnp.dot(xt, wih_ref[...], preferred_element_type=F32) + bih_ref[...]
            gh = jnp.dot(h, whh_ref[...], preferred_element_type=F32) + bhh_ref[...]
            r = jax.nn.sigmoid(gx[:, :H] + gh[:, :H])
            z = jax.nn.sigmoid(gx[:, H : 2 * H] + gh[:, H : 2 * H])
            n = jnp.tanh(gx[:, 2 * H :] + r * gh[:, 2 * H :])
            h = (1.0 - z) * n + z * h
        p = jnp.sum(h * wh_ref[...], axis=1, keepdims=True) + bh_ref[0, 0]
        o_ref[...] = jnp.broadcast_to(p, (block_rows, H))

    full = lambda i: (0, 0)
    return pl.pallas_call(
        body,
        grid=(nb,),
        in_specs=[
            pl.BlockSpec((T, block_rows, H), lambda i: (0, i, 0)),
            pl.BlockSpec((H, 3 * H), full),
            pl.BlockSpec((H, 3 * H), full),
            pl.BlockSpec((1, 3 * H), full),
            pl.BlockSpec((1, 3 * H), full),
            pl.BlockSpec((1, H), full),
            pl.BlockSpec((1, 1), full),
        ],
        out_specs=pl.BlockSpec((block_rows, H), lambda i: (i, 0)),
        out_shape=jax.ShapeDtypeStruct((Np, H), F32),
    )(seq, wih_t, whh_t, bih, bhh, w_head, b_head)


def _message(x_pk, src, dst, e_pk, zeros_blk, t_base, N, H):
    E = src.shape[0]
    HW = H // 2
    NSUB = 16
    NCORE = 2
    CH = 80
    GCH = E // CH
    NCH = (GCH + NSUB - 1) // NSUB
    NCH = NCH + (NCH & 1)
    NP = ((N + 127) // 128) * 128
    NPS = NP // NSUB
    DUMP = NP - 8

    mesh = plsc.VectorSubcoreMesh(
        core_axis_name="c", subcore_axis_name="s", num_cores=NCORE, num_subcores=NSUB
    )

    @functools.partial(
        pl.kernel,
        out_type=jax.ShapeDtypeStruct((NCORE * NP, H), F32),
        mesh=mesh,
        compiler_params=pltpu.CompilerParams(
            needs_layout_passes=False, use_tc_tiling_on_sc=False
        ),
        scratch_types=[
            [pltpu.VMEM((CH,), jnp.int32)] * 2,
            [pltpu.VMEM((CH,), jnp.int32)] * 2,
            [pltpu.VMEM((CH,), jnp.int32)] * 2,
            [pltpu.VMEM((CH, HW), jnp.int32)] * 2,
            [pltpu.VMEM((CH, HW), jnp.int32)] * 2,
            [pltpu.VMEM((CH, H), F32)] * 2,
            pltpu.VMEM_SHARED((NP, H), F32),
            [pltpu.SemaphoreType.DMA] * 2,
            [pltpu.SemaphoreType.DMA] * 2,
            [pltpu.SemaphoreType.DMA] * 2,
        ],
    )
    def msg(x_hbm, src_hbm, dst_hbm, e_hbm, z_hbm, out_hbm, src_v, dst_v, dsc_v,
            e_v, xr_v, xf_v, acc, fsem, gsem, ssem):
        c = lax.axis_index("c")
        s = lax.axis_index("s")

        def chunk_off(k):
            g = k * NSUB + s
            g = jnp.minimum(g, GCH - 1)
            return g * CH

        def fetch(k, b):
            off = chunk_off(k)
            pltpu.async_copy(src_hbm.at[pl.ds(off, CH)], src_v[b], fsem[b])
            pltpu.async_copy(dst_hbm.at[pl.ds(off, CH)], dst_v[b], fsem[b])
            pltpu.async_copy(e_hbm.at[pl.ds(off, CH)], e_v[b], fsem[b])

        def wait_fetch(b):
            pltpu.make_async_copy(src_hbm.at[pl.ds(0, CH)], src_v[b], fsem[b]).wait()
            pltpu.make_async_copy(dst_hbm.at[pl.ds(0, CH)], dst_v[b], fsem[b]).wait()
            pltpu.make_async_copy(e_hbm.at[pl.ds(0, CH)], e_v[b], fsem[b]).wait()

        def prep_idx(k, b, t):
            fake = (k * NSUB + s) >= GCH
            for g in range(CH // 16):
                sl = pl.ds(g * 16, 16)
                src_v[b][sl] = src_v[b][sl] + t * N
                dst_v[b][sl] = jnp.where(fake, DUMP, dst_v[b][sl])

        def gather(b):
            pltpu.async_copy(x_hbm.at[src_v[b]], xr_v[b], gsem[b])

        def wait_gather(b):
            pltpu.make_async_copy(x_hbm.at[src_v[b]], xr_v[b], gsem[b]).wait()

        def wait_scatter(b):
            pltpu.make_async_copy(xf_v[b], acc.at[dsc_v[b]], ssem[b]).wait()

        if True:
            t = t_base + c
            pltpu.sync_copy(z_hbm, acc.at[pl.ds(s * NPS, NPS)])
            plsc.subcore_barrier()

            fetch(0, 0)
            fetch(1, 1)
            wait_fetch(0)
            prep_idx(0, 0, t)
            gather(0)

            def step(ci2, _):
                for b in (0, 1):
                    ci = ci2 * 2 + b
                    nb = 1 - b

                    @pl.when(ci + 1 < NCH)
                    def _():
                        wait_fetch(nb)
                        prep_idx(ci + 1, nb, t)

                    @pl.when(ci >= 1)
                    def _():
                        wait_scatter(nb)

                    @pl.when(ci + 1 < NCH)
                    def _():
                        gather(nb)

                    wait_gather(b)

                    @plsc.parallel_loop(0, CH, unroll=8)
                    def row(i):
                        for g in range(HW // 16):
                            sl = pl.ds(g * 16, 16)
                            xb = plsc.bitcast(xr_v[b][i, sl], jnp.bfloat16)
                            eb = plsc.bitcast(e_v[b][i, sl], jnp.bfloat16)
                            m = jnp.maximum(xb + eb, jnp.zeros((32,), jnp.bfloat16))
                            w = plsc.bitcast(m, jnp.int32)
                            xf_v[b][i, pl.ds(g * 16, 16)] = plsc.bitcast(
                                w << 16, F32
                            )
                            xf_v[b][i, pl.ds(HW + g * 16, 16)] = plsc.bitcast(
                                w & jnp.int32(-65536), F32
                            )
                    for g in range(CH // 16):
                        sl = pl.ds(g * 16, 16)
                        dsc_v[b][sl] = dst_v[b][sl]
                    pltpu.async_copy(xf_v[b], acc.at[dsc_v[b]], ssem[b], add=True)

                    @pl.when(ci + 2 < NCH)
                    def _():
                        fetch(ci + 2, b)

                return 0

            lax.fori_loop(0, NCH // 2, step, 0)
            wait_scatter(1)
            plsc.subcore_barrier()
            pltpu.sync_copy(
                acc.at[pl.ds(s * NPS, NPS)],
                out_hbm.at[pl.ds(c * NP + s * NPS, NPS)],
            )

    NPpad = ((N + 127) // 128) * 128
    out = msg(x_pk, src, dst, e_pk, zeros_blk)
    return out.reshape(2, NPpad, H)


def kernel(x_seq, edge_index, edge_attr, W_enc, b_enc, lin0_W, lin0_b, mlp0_W1,
           mlp0_b1, mlp0_W2, mlp0_b2, ln0_g, ln0_b, lin1_W, lin1_b, mlp1_W1,
           mlp1_b1, mlp1_W2, mlp1_b2, ln1_g, ln1_b, W_ih, W_hh, b_ih, b_hh,
           W_head, b_head):
    B, T, N, F = x_seq.shape
    H = W_enc.shape[0]
    src = edge_index[0]
    dst = edge_index[1]

    r2 = lambda v: v.reshape(1, -1)

    e0p, e1p = _edge_embed(edge_attr, lin0_W.T, r2(lin0_b), lin1_W.T, r2(lin1_b))
    X, Xp = _encode(x_seq.reshape(T * N, F), W_enc.T, r2(b_enc))

    zeros_blk = jnp.zeros((((N + 127) // 128) * 128 // 16, H), F32)

    agg0_a = _message(Xp, src, dst, e0p, zeros_blk, 0, N, H)
    agg0_b = _message(Xp, src, dst, e0p, zeros_blk, 2, N, H)
    post0 = lambda xs, ag, ofs: _post(
        xs, ag, mlp0_W1.T, r2(mlp0_b1), mlp0_W2.T, r2(mlp0_b2), r2(ln0_g),
        r2(ln0_b), emit_packed=True, x_ofs=ofs,
    )
    X1a, X1pa = post0(X, agg0_a, 0)
    X1b, X1pb = post0(X, agg0_b, 20)
    agg1_a = _message(X1pa, src, dst, e1p, zeros_blk, 0, N, H)
    agg1_b = _message(X1pb, src, dst, e1p, zeros_blk, 0, N, H)
    post1 = lambda xs, ag: _post(
        xs, ag, mlp1_W1.T, r2(mlp1_b1), mlp1_W2.T, r2(mlp1_b2), r2(ln1_g),
        r2(ln1_b), emit_packed=False,
    )[0]
    X2 = jnp.concatenate([post1(X1a, agg1_a), post1(X1b, agg1_b)], axis=0)

    seq = X2.reshape(T, N, H)
    hout = _gru_head(seq, W_ih.T, W_hh.T, r2(b_ih), r2(b_hh), W_head,
                     b_head.reshape(1, 1), block_rows=1000)
    return hout[:, 0].reshape(1, N)


# speedup vs baseline: 1.9490x; 1.0088x over previous
import functools

import jax
import jax.numpy as jnp
from jax import lax
from jax.experimental import pallas as pl
from jax.experimental.pallas import tpu as pltpu
from jax.experimental.pallas import tpu_sc as plsc

F32 = jnp.float32
U32 = jnp.uint32


def _pack_bf16_pairs(y):
    yi = lax.bitcast_convert_type(y, U32)
    lsb = (yi >> 16) & jnp.uint32(1)
    r = (yi + jnp.uint32(0x7FFF) + lsb) >> 16
    half = y.shape[-1] // 2
    pk = r[:, :half] | (r[:, half:] << 16)
    return lax.bitcast_convert_type(pk, jnp.int32)


def _encode(x, w_t, b, block_rows=2000):
    M, K = x.shape
    Hout = w_t.shape[1]
    nb = M // block_rows

    def body(x_ref, w_ref, b_ref, o_ref, op_ref):
        y = jnp.dot(x_ref[...], w_ref[...], preferred_element_type=F32) + b_ref[...]
        o_ref[...] = y
        op_ref[...] = _pack_bf16_pairs(y)

    full = lambda i: (0, 0)
    return pl.pallas_call(
        body,
        grid=(nb,),
        in_specs=[
            pl.BlockSpec((block_rows, K), lambda i: (i, 0)),
            pl.BlockSpec((K, Hout), full),
            pl.BlockSpec((1, Hout), full),
        ],
        out_specs=[
            pl.BlockSpec((block_rows, Hout), lambda i: (i, 0)),
            pl.BlockSpec((block_rows, Hout // 2), lambda i: (i, 0)),
        ],
        out_shape=[
            jax.ShapeDtypeStruct((M, Hout), F32),
            jax.ShapeDtypeStruct((M, Hout // 2), jnp.int32),
        ],
    )(x, w_t, b)


def _edge_embed(attr, w0_t, b0, w1_t, b1, block_rows=2000):
    E, D = attr.shape
    H = w0_t.shape[1]
    nb = E // block_rows

    def body(a_ref, w0_ref, b0_ref, w1_ref, b1_ref, e0_ref, e1_ref):
        a = a_ref[...]
        e0_ref[...] = _pack_bf16_pairs(
            jnp.dot(a, w0_ref[...], preferred_element_type=F32) + b0_ref[...]
        )
        e1_ref[...] = _pack_bf16_pairs(
            jnp.dot(a, w1_ref[...], preferred_element_type=F32) + b1_ref[...]
        )

    full = lambda i: (0, 0)
    return pl.pallas_call(
        body,
        grid=(nb,),
        in_specs=[
            pl.BlockSpec((block_rows, D), lambda i: (i, 0)),
            pl.BlockSpec((D, H), full),
            pl.BlockSpec((1, H), full),
            pl.BlockSpec((D, H), full),
            pl.BlockSpec((1, H), full),
        ],
        out_specs=[
            pl.BlockSpec((block_rows, H // 2), lambda i: (i, 0)),
            pl.BlockSpec((block_rows, H // 2), lambda i: (i, 0)),
        ],
        out_shape=[
            jax.ShapeDtypeStruct((E, H // 2), jnp.int32),
            jax.ShapeDtypeStruct((E, H // 2), jnp.int32),
        ],
    )(attr, w0_t, b0, w1_t, b1)


def _post(x, agg3, w1_t, b1, w2_t, b2, g, b, emit_packed, x_ofs=0,
          block_rows=1000):
    _, NP, H = agg3.shape
    M = 2 * (block_rows * (NP // block_rows))
    npb = NP // block_rows
    nb = M // block_rows
    full = lambda i: (0, 0)
    blk = lambda i: (i, 0)

    def body(x_ref, a_ref, w1_ref, b1_ref, w2_ref, b2_ref, g_ref, bb_ref,
             o_ref, *op_ref):
        x_ = x_ref[...]
        h = x_ + a_ref[0]
        y = jnp.maximum(
            jnp.dot(h, w1_ref[...], preferred_element_type=F32) + b1_ref[...], 0.0
        )
        y = jnp.dot(y, w2_ref[...], preferred_element_type=F32) + b2_ref[...]
        mu = jnp.mean(y, axis=-1, keepdims=True)
        var = jnp.mean((y - mu) ** 2, axis=-1, keepdims=True)
        z = (y - mu) * lax.rsqrt(var + 1e-5) * g_ref[...] + bb_ref[...]
        o = jnp.maximum(z, 0.0) + x_
        o_ref[...] = o
        if op_ref:
            op_ref[0][...] = _pack_bf16_pairs(o)

    out_specs = [pl.BlockSpec((block_rows, H), blk)]
    out_shape = [jax.ShapeDtypeStruct((M, H), F32)]
    if emit_packed:
        out_specs.append(pl.BlockSpec((block_rows, H // 2), blk))
        out_shape.append(jax.ShapeDtypeStruct((M, H // 2), jnp.int32))

    res = pl.pallas_call(
        body,
        grid=(nb,),
        in_specs=[
            pl.BlockSpec((block_rows, H), lambda i: (x_ofs + i, 0)),
            pl.BlockSpec((1, block_rows, H), lambda i: (i // npb, i % npb, 0)),
        ]
        + [
            pl.BlockSpec((H, H), full),
            pl.BlockSpec((1, H), full),
            pl.BlockSpec((H, H), full),
            pl.BlockSpec((1, H), full),
            pl.BlockSpec((1, H), full),
            pl.BlockSpec((1, H), full),
        ],
        out_specs=out_specs,
        out_shape=out_shape,
    )(x, agg3, w1_t, b1, w2_t, b2, g, b)
    return res if emit_packed else (res[0], None)


def _gru_head(seq_a, seq_b, wih_t, whh_t, bih, bhh, w_head, b_head,
              block_rows=1000):
    _, Np, H = seq_a.shape
    T = 4
    nb = Np // block_rows

    def body(sa_ref, sb_ref, wih_ref, whh_ref, bih_ref, bhh_ref, wh_ref,
             bh_ref, o_ref):
        h = jnp.zeros((block_rows, H), F32)
        for t in range(T):
            xt = sa_ref[t] if t < 2 else sb_ref[t - 2]
            gx = jnp.dot(xt, wih_ref[...], preferred_element_type=F32) + bih_ref[...]
            gh = jnp.dot(h, whh_ref[...], preferred_element_type=F32) + bhh_ref[...]
            r = jax.nn.sigmoid(gx[:, :H] + gh[:, :H])
            z = jax.nn.sigmoid(gx[:, H : 2 * H] + gh[:, H : 2 * H])
            n = jnp.tanh(gx[:, 2 * H :] + r * gh[:, 2 * H :])
            h = (1.0 - z) * n + z * h
        p = jnp.sum(h * wh_ref[...], axis=1, keepdims=True) + bh_ref[0, 0]
        o_ref[...] = jnp.broadcast_to(p, (block_rows, H))

    full = lambda i: (0, 0)
    return pl.pallas_call(
        body,
        grid=(nb,),
        in_specs=[
            pl.BlockSpec((2, block_rows, H), lambda i: (0, i, 0)),
            pl.BlockSpec((2, block_rows, H), lambda i: (0, i, 0)),
            pl.BlockSpec((H, 3 * H), full),
            pl.BlockSpec((H, 3 * H), full),
            pl.BlockSpec((1, 3 * H), full),
            pl.BlockSpec((1, 3 * H), full),
            pl.BlockSpec((1, H), full),
            pl.BlockSpec((1, 1), full),
        ],
        out_specs=pl.BlockSpec((block_rows, H), lambda i: (i, 0)),
        out_shape=jax.ShapeDtypeStruct((Np, H), F32),
    )(seq_a, seq_b, wih_t, whh_t, bih, bhh, w_head, b_head)


def _message(x_pk, src, dst, e_pk, zeros_blk, t_base, N, H):
    E = src.shape[0]
    HW = H // 2
    NSUB = 16
    NCORE = 2
    CH = 80
    GCH = E // CH
    NCH = (GCH + NSUB - 1) // NSUB
    NCH = NCH + (NCH & 1)
    NP = ((N + 127) // 128) * 128
    NPS = NP // NSUB
    DUMP = NP - 8

    mesh = plsc.VectorSubcoreMesh(
        core_axis_name="c", subcore_axis_name="s", num_cores=NCORE, num_subcores=NSUB
    )

    @functools.partial(
        pl.kernel,
        out_type=jax.ShapeDtypeStruct((NCORE * NP, H), F32),
        mesh=mesh,
        compiler_params=pltpu.CompilerParams(
            needs_layout_passes=False, use_tc_tiling_on_sc=False
        ),
        scratch_types=[
            [pltpu.VMEM((CH,), jnp.int32)] * 2,
            [pltpu.VMEM((CH,), jnp.int32)] * 2,
            [pltpu.VMEM((CH,), jnp.int32)] * 2,
            [pltpu.VMEM((CH, HW), jnp.int32)] * 2,
            [pltpu.VMEM((CH, HW), jnp.int32)] * 2,
            [pltpu.VMEM((CH, H), F32)] * 2,
            pltpu.VMEM_SHARED((NP, H), F32),
            [pltpu.SemaphoreType.DMA] * 2,
            [pltpu.SemaphoreType.DMA] * 2,
            [pltpu.SemaphoreType.DMA] * 2,
        ],
    )
    def msg(x_hbm, src_hbm, dst_hbm, e_hbm, z_hbm, out_hbm, src_v, dst_v, dsc_v,
            e_v, xr_v, xf_v, acc, fsem, gsem, ssem):
        c = lax.axis_index("c")
        s = lax.axis_index("s")

        def chunk_off(k):
            g = k * NSUB + s
            g = jnp.minimum(g, GCH - 1)
            return g * CH

        def fetch(k, b):
            off = chunk_off(k)
            pltpu.async_copy(src_hbm.at[pl.ds(off, CH)], src_v[b], fsem[b])
            pltpu.async_copy(dst_hbm.at[pl.ds(off, CH)], dst_v[b], fsem[b])
            pltpu.async_copy(e_hbm.at[pl.ds(off, CH)], e_v[b], fsem[b])

        def wait_fetch(b):
            pltpu.make_async_copy(src_hbm.at[pl.ds(0, CH)], src_v[b], fsem[b]).wait()
            pltpu.make_async_copy(dst_hbm.at[pl.ds(0, CH)], dst_v[b], fsem[b]).wait()
            pltpu.make_async_copy(e_hbm.at[pl.ds(0, CH)], e_v[b], fsem[b]).wait()

        def prep_idx(k, b, t):
            fake = (k * NSUB + s) >= GCH
            for g in range(CH // 16):
                sl = pl.ds(g * 16, 16)
                src_v[b][sl] = src_v[b][sl] + t * N
                dst_v[b][sl] = jnp.where(fake, DUMP, dst_v[b][sl])

        def gather(b):
            pltpu.async_copy(x_hbm.at[src_v[b]], xr_v[b], gsem[b])

        def wait_gather(b):
            pltpu.make_async_copy(x_hbm.at[src_v[b]], xr_v[b], gsem[b]).wait()

        def wait_scatter(b):
            pltpu.make_async_copy(xf_v[b], acc.at[dsc_v[b]], ssem[b]).wait()

        if True:
            t = t_base + c
            pltpu.sync_copy(z_hbm, acc.at[pl.ds(s * NPS, NPS)])
            plsc.subcore_barrier()

            fetch(0, 0)
            fetch(1, 1)
            wait_fetch(0)
            prep_idx(0, 0, t)
            gather(0)

            def step(ci2, _):
                for b in (0, 1):
                    ci = ci2 * 2 + b
                    nb = 1 - b

                    @pl.when(ci + 1 < NCH)
                    def _():
                        wait_fetch(nb)
                        prep_idx(ci + 1, nb, t)

                    @pl.when(ci >= 1)
                    def _():
                        wait_scatter(nb)

                    @pl.when(ci + 1 < NCH)
                    def _():
                        gather(nb)

                    wait_gather(b)

                    @plsc.parallel_loop(0, CH, unroll=8)
                    def row(i):
                        for g in range(HW // 16):
                            sl = pl.ds(g * 16, 16)
                            xb = plsc.bitcast(xr_v[b][i, sl], jnp.bfloat16)
                            eb = plsc.bitcast(e_v[b][i, sl], jnp.bfloat16)
                            m = jnp.maximum(xb + eb, jnp.zeros((32,), jnp.bfloat16))
                            w = plsc.bitcast(m, jnp.int32)
                            xf_v[b][i, pl.ds(g * 16, 16)] = plsc.bitcast(
                                w << 16, F32
                            )
                            xf_v[b][i, pl.ds(HW + g * 16, 16)] = plsc.bitcast(
                                w & jnp.int32(-65536), F32
                            )
                    for g in range(CH // 16):
                        sl = pl.ds(g * 16, 16)
                        dsc_v[b][sl] = dst_v[b][sl]
                    pltpu.async_copy(xf_v[b], acc.at[dsc_v[b]], ssem[b], add=True)

                    @pl.when(ci + 2 < NCH)
                    def _():
                        fetch(ci + 2, b)

                return 0

            lax.fori_loop(0, NCH // 2, step, 0)
            wait_scatter(1)
            plsc.subcore_barrier()
            pltpu.sync_copy(
                acc.at[pl.ds(s * NPS, NPS)],
                out_hbm.at[pl.ds(c * NP + s * NPS, NPS)],
            )

    NPpad = ((N + 127) // 128) * 128
    out = msg(x_pk, src, dst, e_pk, zeros_blk)
    return out.reshape(2, NPpad, H)


def kernel(x_seq, edge_index, edge_attr, W_enc, b_enc, lin0_W, lin0_b, mlp0_W1,
           mlp0_b1, mlp0_W2, mlp0_b2, ln0_g, ln0_b, lin1_W, lin1_b, mlp1_W1,
           mlp1_b1, mlp1_W2, mlp1_b2, ln1_g, ln1_b, W_ih, W_hh, b_ih, b_hh,
           W_head, b_head):
    B, T, N, F = x_seq.shape
    H = W_enc.shape[0]
    src = edge_index[0]
    dst = edge_index[1]

    r2 = lambda v: v.reshape(1, -1)

    e0p, e1p = _edge_embed(edge_attr, lin0_W.T, r2(lin0_b), lin1_W.T, r2(lin1_b))
    X, Xp = _encode(x_seq.reshape(T * N, F), W_enc.T, r2(b_enc))

    zeros_blk = jnp.zeros((((N + 127) // 128) * 128 // 16, H), F32)

    agg0_a = _message(Xp, src, dst, e0p, zeros_blk, 0, N, H)
    agg0_b = _message(Xp, src, dst, e0p, zeros_blk, 2, N, H)
    post0 = lambda xs, ag, ofs: _post(
        xs, ag, mlp0_W1.T, r2(mlp0_b1), mlp0_W2.T, r2(mlp0_b2), r2(ln0_g),
        r2(ln0_b), emit_packed=True, x_ofs=ofs,
    )
    X1a, X1pa = post0(X, agg0_a, 0)
    X1b, X1pb = post0(X, agg0_b, 20)
    agg1_a = _message(X1pa, src, dst, e1p, zeros_blk, 0, N, H)
    agg1_b = _message(X1pb, src, dst, e1p, zeros_blk, 0, N, H)
    post1 = lambda xs, ag: _post(
        xs, ag, mlp1_W1.T, r2(mlp1_b1), mlp1_W2.T, r2(mlp1_b2), r2(ln1_g),
        r2(ln1_b), emit_packed=False,
    )[0]
    X2a = post1(X1a, agg1_a).reshape(2, N, H)
    X2b = post1(X1b, agg1_b).reshape(2, N, H)

    hout = _gru_head(X2a, X2b, W_ih.T, W_hh.T, r2(b_ih), r2(b_hh), W_head,
                     b_head.reshape(1, 1))
    return hout[:, 0].reshape(1, N)
